# baseline (speedup 1.0000x reference)
import functools

import jax
import jax.numpy as jnp
from jax import lax
from jax.experimental import pallas as pl
from jax.experimental.pallas import tpu as pltpu
from jax.experimental.pallas import tpu_sc as plsc

N = 10000
E = 160000
D_IN = 256
HID = 20
HP = 32
D_OUT = 128
NN = 2 * N

NC, NS = 2, 16
NW = NC * NS
CH = 128
E2 = 2 * E
NBUF = 8
K = -(-E2 // (NW * CH * NBUF)) * NBUF
KP = K + NBUF
EP = NW * K * CH
EPAD = EP - E2

ACC_ROWS = 20480
ZROWS = 160
ACC_PER_SUB = ACC_ROWS // NS


def _seg_body(y_hbm, src_hbm, dst_hbm, p_hbm, acc_s, src_v, dst_v, rows_v,
              zbuf_v, sem_g, sem_s):
    c = lax.axis_index("c")
    s = lax.axis_index("s")
    wid = c * NS + s

    pltpu.sync_copy(src_hbm.at[wid], src_v)
    pltpu.sync_copy(dst_hbm.at[wid], dst_v)

    zeros16 = jnp.zeros((16,), jnp.float32)

    def _zfill(i, carry):
        zbuf_v[i // 2, pl.ds((i % 2) * 16, 16)] = zeros16
        return carry

    lax.fori_loop(0, ZROWS * 2, _zfill, 0)
    for t in range(ACC_PER_SUB // ZROWS):
        pltpu.sync_copy(zbuf_v,
                        acc_s.at[pl.ds(s * ACC_PER_SUB + t * ZROWS, ZROWS)])
    plsc.subcore_barrier()

    for b in range(NBUF):
        pltpu.async_copy(y_hbm.at[src_v.at[b]], rows_v.at[b], sem_g)

    def _group(g, carry):
        j0 = g * NBUF
        for b in range(NBUF):
            pltpu.make_async_copy(y_hbm.at[src_v.at[j0 + b]], rows_v.at[b],
                                  sem_g).wait()
        for b in range(NBUF):
            pltpu.async_copy(rows_v.at[b], acc_s.at[dst_v.at[j0 + b]], sem_s,
                             add=True)
        for b in range(NBUF):
            pltpu.make_async_copy(rows_v.at[b], acc_s.at[dst_v.at[j0 + b]],
                                  sem_s).wait()
        for b in range(NBUF):
            pltpu.async_copy(y_hbm.at[src_v.at[j0 + NBUF + b]], rows_v.at[b],
                             sem_g)
        return carry

    lax.fori_loop(0, K // NBUF, _group, 0)
    for b in range(NBUF):
        pltpu.make_async_copy(y_hbm.at[src_v.at[b]], rows_v.at[b],
                              sem_g).wait()
    plsc.subcore_barrier()

    pltpu.sync_copy(acc_s.at[pl.ds(s * ACC_PER_SUB, ACC_PER_SUB)],
                    p_hbm.at[c, pl.ds(s * ACC_PER_SUB, ACC_PER_SUB)])


@functools.lru_cache(maxsize=1)
def _seg_kernel():
    return pl.kernel(
        _seg_body,
        out_type=jax.ShapeDtypeStruct((NC, ACC_ROWS, HP), jnp.float32),
        mesh=plsc.VectorSubcoreMesh(core_axis_name="c", subcore_axis_name="s",
                                    num_cores=NC, num_subcores=NS),
        scratch_types=[
            pltpu.VMEM_SHARED((ACC_ROWS, HP), jnp.float32),
            pltpu.VMEM((KP, CH), jnp.int32),
            pltpu.VMEM((KP, CH), jnp.int32),
            pltpu.VMEM((NBUF, CH, HP), jnp.float32),
            pltpu.VMEM((ZROWS, HP), jnp.float32),
            pltpu.SemaphoreType.DMA,
            pltpu.SemaphoreType.DMA,
        ],
        compiler_params=pltpu.CompilerParams(use_tc_tiling_on_sc=False),
    )


def _seg(y, src3, dst3):
    return _seg_kernel()(y, src3, dst3)


_BLK_A = 2000


def _stage_a_body(x_ref, wr_ref, wq_ref, t_ref, r_ref):
    x = x_ref[...]
    t_ref[...] = jnp.dot(x, wr_ref[...], preferred_element_type=jnp.float32)
    r_ref[...] = jnp.dot(x, wq_ref[...], preferred_element_type=jnp.float32)


def _stage_a(x, wr, wq):
    grid = (NN // _BLK_A,)
    return pl.pallas_call(
        _stage_a_body,
        grid=grid,
        in_specs=[
            pl.BlockSpec((_BLK_A, D_IN), lambda i: (i, 0)),
            pl.BlockSpec((D_IN, HP), lambda i: (0, 0)),
            pl.BlockSpec((D_IN, HP), lambda i: (0, 0)),
        ],
        out_specs=[pl.BlockSpec((_BLK_A, HP), lambda i: (i, 0))] * 2,
        out_shape=[jax.ShapeDtypeStruct((NN, HP), jnp.float32)] * 2,
    )(x, wr, wq)


_BLK_B = 2000


def _stage_b1_body(p_ref, a_ref, b_ref, wr_ref, h_ref, t_ref):
    h = jnp.maximum(p_ref[0] + p_ref[1] + b_ref[...] + a_ref[...], 0.0)
    h_ref[...] = h
    t_ref[...] = jnp.dot(h, wr_ref[...], preferred_element_type=jnp.float32)


def _stage_b2_body(p_ref, a_ref, b_ref, wq_ref, wr_ref, h_ref, t_ref):
    root = jnp.dot(a_ref[...], wq_ref[...], preferred_element_type=jnp.float32)
    h = jnp.maximum(p_ref[0] + p_ref[1] + b_ref[...] + root, 0.0)
    h_ref[...] = h
    t_ref[...] = jnp.dot(h, wr_ref[...], preferred_element_type=jnp.float32)


def _stage_b3_body(p_ref, a_ref, b_ref, wq_ref, h_ref):
    root = jnp.dot(a_ref[...], wq_ref[...], preferred_element_type=jnp.float32)
    h_ref[...] = jnp.maximum(p_ref[0] + p_ref[1] + b_ref[...] + root, 0.0)


def _stage_b1(p, a, b, wr):
    grid = (NN // _BLK_B,)
    return pl.pallas_call(
        _stage_b1_body,
        grid=grid,
        in_specs=[
            pl.BlockSpec((NC, _BLK_B, HP), lambda i: (0, i, 0)),
            pl.BlockSpec((_BLK_B, HP), lambda i: (i, 0)),
            pl.BlockSpec((1, HP), lambda i: (0, 0)),
            pl.BlockSpec((HP, HP), lambda i: (0, 0)),
        ],
        out_specs=[pl.BlockSpec((_BLK_B, HP), lambda i: (i, 0))] * 2,
        out_shape=[jax.ShapeDtypeStruct((NN, HP), jnp.float32)] * 2,
    )(p, a, b, wr)


def _stage_b2(p, a, b, wq, wr):
    grid = (NN // _BLK_B,)
    return pl.pallas_call(
        _stage_b2_body,
        grid=grid,
        in_specs=[
            pl.BlockSpec((NC, _BLK_B, HP), lambda i: (0, i, 0)),
            pl.BlockSpec((_BLK_B, HP), lambda i: (i, 0)),
            pl.BlockSpec((1, HP), lambda i: (0, 0)),
            pl.BlockSpec((HP, HP), lambda i: (0, 0)),
            pl.BlockSpec((HP, HP), lambda i: (0, 0)),
        ],
        out_specs=[pl.BlockSpec((_BLK_B, HP), lambda i: (i, 0))] * 2,
        out_shape=[jax.ShapeDtypeStruct((NN, HP), jnp.float32)] * 2,
    )(p, a, b, wq, wr)


def _stage_b3(p, a, b, wq):
    grid = (NN // _BLK_B,)
    return pl.pallas_call(
        _stage_b3_body,
        grid=grid,
        in_specs=[
            pl.BlockSpec((NC, _BLK_B, HP), lambda i: (0, i, 0)),
            pl.BlockSpec((_BLK_B, HP), lambda i: (i, 0)),
            pl.BlockSpec((1, HP), lambda i: (0, 0)),
            pl.BlockSpec((HP, HP), lambda i: (0, 0)),
        ],
        out_specs=pl.BlockSpec((_BLK_B, HP), lambda i: (i, 0)),
        out_shape=jax.ShapeDtypeStruct((NN, HP), jnp.float32),
    )(p, a, b, wq)


_BLK_C = 1000
_NBLK_C = NN // _BLK_C
_MOL_BLKS = N // _BLK_C


def _stage_c_body(p_ref, h3_ref, wr_ref, bo_ref, wq_ref, wl1_ref, bl1_ref,
                  wl2_ref, bl2_ref, wh1_ref, bh1_ref, wh2_ref, bh2_ref,
                  wh3_ref, bh3_ref, out_ref, acc):
    i = pl.program_id(0)
    agg = p_ref[0] + p_ref[1]
    h4 = jnp.maximum(
        jnp.dot(agg, wr_ref[...], preferred_element_type=jnp.float32)
        + bo_ref[...]
        + jnp.dot(h3_ref[...], wq_ref[...], preferred_element_type=jnp.float32),
        0.0,
    )
    bs = jnp.sum(h4, axis=0, keepdims=True)

    @pl.when(i == 0)
    def _():
        acc[0:1, :] = bs

    @pl.when((i > 0) & (i < _MOL_BLKS))
    def _():
        acc[0:1, :] = acc[0:1, :] + bs

    @pl.when(i == _MOL_BLKS)
    def _():
        acc[1:2, :] = bs

    @pl.when(i > _MOL_BLKS)
    def _():
        acc[1:2, :] = acc[1:2, :] + bs

    @pl.when(i == _NBLK_C - 1)
    def _():
        m = jnp.maximum(
            jnp.dot(acc[...], wl1_ref[...], preferred_element_type=jnp.float32)
            + bl1_ref[...], 0.0)
        m = jnp.maximum(
            jnp.dot(m, wl2_ref[...], preferred_element_type=jnp.float32)
            + bl2_ref[...], 0.0)
        z = (jnp.dot(m[0:1, :], wh1_ref[0:D_OUT, :],
                     preferred_element_type=jnp.float32)
             + jnp.dot(m[1:2, :], wh1_ref[D_OUT:2 * D_OUT, :],
                       preferred_element_type=jnp.float32)
             + bh1_ref[...])
        z = jnp.maximum(z, 0.0)
        z = jnp.maximum(
            jnp.dot(z, wh2_ref[...], preferred_element_type=jnp.float32)
            + bh2_ref[...], 0.0)
        z = (jnp.dot(z, wh3_ref[...], preferred_element_type=jnp.float32)
             + bh3_ref[...])
        out_ref[...] = 1.0 / (1.0 + jnp.exp(-z))


def _stage_c(p, h3, wr, bo, wq, wl1, bl1, wl2, bl2, wh1, bh1, wh2, bh2, wh3,
             bh3):
    grid = (_NBLK_C,)

    def _full(shape):
        nd = len(shape)
        return pl.BlockSpec(shape, lambda i, _nd=nd: (0,) * _nd)

    return pl.pallas_call(
        _stage_c_body,
        grid=grid,
        in_specs=[
            pl.BlockSpec((NC, _BLK_C, HP), lambda i: (0, i, 0)),
            pl.BlockSpec((_BLK_C, HP), lambda i: (i, 0)),
            _full((HP, D_OUT)),
            _full((1, D_OUT)),
            _full((HP, D_OUT)),
            _full((D_OUT, D_OUT)),
            _full((1, D_OUT)),
            _full((D_OUT, D_OUT)),
            _full((1, D_OUT)),
            _full((2 * D_OUT, 10)),
            _full((1, 10)),
            _full((10, 10)),
            _full((1, 10)),
            _full((10, 1)),
            _full((1, 1)),
        ],
        out_specs=pl.BlockSpec((1, 1), lambda i: (0, 0)),
        out_shape=jax.ShapeDtypeStruct((1, 1), jnp.float32),
        scratch_shapes=[pltpu.VMEM((2, D_OUT), jnp.float32)],
    )(p, h3, wr, bo, wq, wl1, bl1, wl2, bl2, wh1, bh1, wh2, bh2, wh3, bh3)


def _pad_cols(w, width=HP):
    return jnp.pad(w, ((0, 0), (0, width - w.shape[1])))


def _pad_rows(w, height=HP):
    return jnp.pad(w, ((0, height - w.shape[0]), (0, 0)))


def kernel(mol_1_graph, mol_1_nodes, mol_2_graph, mol_2_nodes, params):
    pr = params
    wr_in = _pad_cols(pr['conv_in']['W_rel'])
    wq_in = _pad_cols(pr['conv_in']['W_root'])
    b_in = _pad_cols(pr['conv_in']['b'][None])
    li1, li2 = pr['conv_internal']
    wr1 = _pad_cols(_pad_rows(li1['W_rel']))
    wq1 = _pad_cols(_pad_rows(li1['W_root']))
    b1 = _pad_cols(li1['b'][None])
    wr2 = _pad_cols(_pad_rows(li2['W_rel']))
    wq2 = _pad_cols(_pad_rows(li2['W_root']))
    b2 = _pad_cols(li2['b'][None])
    wr_out = _pad_rows(pr['conv_out']['W_rel'])
    wq_out = _pad_rows(pr['conv_out']['W_root'])
    b_out = pr['conv_out']['b'][None]
    lo1, lo2 = pr['linear_output']
    wh1 = pr['linear_1']['W']
    bh1 = pr['linear_1']['b'][None]
    wh2 = pr['linear_2']['W']
    bh2 = pr['linear_2']['b'][None]
    wh3 = pr['linear_3']['W']
    bh3 = pr['linear_3']['b'][None]

    x = jnp.concatenate([mol_1_nodes, mol_2_nodes], axis=0)
    src = jnp.concatenate([
        mol_1_graph[0], mol_2_graph[0] + N,
        jnp.zeros((EPAD,), jnp.int32),
    ])
    dst = jnp.concatenate([
        mol_1_graph[1], mol_2_graph[1] + N,
        jnp.full((EPAD,), NN, jnp.int32),
    ])
    src3 = jnp.concatenate([
        src.reshape(NW, K, CH),
        jnp.zeros((NW, KP - K, CH), jnp.int32),
    ], axis=1)
    dst3 = jnp.concatenate([
        dst.reshape(NW, K, CH),
        jnp.full((NW, KP - K, CH), NN, jnp.int32),
    ], axis=1)

    t0, r0 = _stage_a(x, wr_in, wq_in)
    p0 = _seg(t0, src3, dst3)
    h1, t1 = _stage_b1(p0, r0, b_in, wr1)
    p1 = _seg(t1, src3, dst3)
    h2, t2 = _stage_b2(p1, h1, b1, wq1, wr2)
    p2 = _seg(t2, src3, dst3)
    h3 = _stage_b3(p2, h2, b2, wq2)
    p3 = _seg(h3, src3, dst3)
    out = _stage_c(p3, h3, wr_out, b_out, wq_out,
                   lo1['W'], lo1['b'][None], lo2['W'], lo2['b'][None],
                   wh1, bh1, wh2, bh2, wh3, bh3)
    return out.reshape((1,))


# speedup vs baseline: 1.6886x; 1.6886x over previous
import functools

import jax
import jax.numpy as jnp
from jax import lax
from jax.experimental import pallas as pl
from jax.experimental.pallas import tpu as pltpu
from jax.experimental.pallas import tpu_sc as plsc

N = 10000
E = 160000
D_IN = 256
HID = 20
HP = 32
D_OUT = 128
NN = 2 * N

NC, NS = 2, 16
NW = NC * NS
CH = 128
E2 = 2 * E
NBUF = 2
K = -(-E2 // (NW * CH * NBUF)) * NBUF
KP = K + NBUF
EP = NW * K * CH
EPAD = EP - E2

ACC_ROWS = 20480
ZROWS = 160
ACC_PER_SUB = ACC_ROWS // NS


def _seg_body(y_hbm, src_hbm, dst_hbm, p_hbm, acc_s, src_v, dst_v, rows_v,
              zbuf_v, sem_g, sem_s):
    c = lax.axis_index("c")
    s = lax.axis_index("s")
    wid = c * NS + s

    pltpu.sync_copy(src_hbm.at[wid], src_v)
    pltpu.sync_copy(dst_hbm.at[wid], dst_v)

    zeros16 = jnp.zeros((16,), jnp.float32)

    def _zfill(i, carry):
        zbuf_v[i // 2, pl.ds((i % 2) * 16, 16)] = zeros16
        return carry

    lax.fori_loop(0, ZROWS * 2, _zfill, 0)
    for t in range(ACC_PER_SUB // ZROWS):
        pltpu.sync_copy(zbuf_v,
                        acc_s.at[pl.ds(s * ACC_PER_SUB + t * ZROWS, ZROWS)])
    plsc.subcore_barrier()

    for b in range(NBUF):
        pltpu.async_copy(y_hbm.at[src_v.at[b]], rows_v.at[b], sem_g)

    def _group(g, carry):
        j0 = g * NBUF
        for b in range(NBUF):
            pltpu.make_async_copy(y_hbm.at[src_v.at[j0 + b]], rows_v.at[b],
                                  sem_g).wait()
        for b in range(NBUF):
            pltpu.async_copy(rows_v.at[b], acc_s.at[dst_v.at[j0 + b]], sem_s,
                             add=True)
        for b in range(NBUF):
            pltpu.make_async_copy(rows_v.at[b], acc_s.at[dst_v.at[j0 + b]],
                                  sem_s).wait()
        for b in range(NBUF):
            pltpu.async_copy(y_hbm.at[src_v.at[j0 + NBUF + b]], rows_v.at[b],
                             sem_g)
        return carry

    lax.fori_loop(0, K // NBUF, _group, 0)
    for b in range(NBUF):
        pltpu.make_async_copy(y_hbm.at[src_v.at[b]], rows_v.at[b],
                              sem_g).wait()
    plsc.subcore_barrier()

    pltpu.sync_copy(acc_s.at[pl.ds(s * ACC_PER_SUB, ACC_PER_SUB)],
                    p_hbm.at[c, pl.ds(s * ACC_PER_SUB, ACC_PER_SUB)])


@functools.lru_cache(maxsize=1)
def _seg_kernel():
    return pl.kernel(
        _seg_body,
        out_type=jax.ShapeDtypeStruct((NC, ACC_ROWS, HP), jnp.float32),
        mesh=plsc.VectorSubcoreMesh(core_axis_name="c", subcore_axis_name="s",
                                    num_cores=NC, num_subcores=NS),
        scratch_types=[
            pltpu.VMEM_SHARED((ACC_ROWS, HP), jnp.float32),
            pltpu.VMEM((KP, CH), jnp.int32),
            pltpu.VMEM((KP, CH), jnp.int32),
            pltpu.VMEM((NBUF, CH, HP), jnp.float32),
            pltpu.VMEM((ZROWS, HP), jnp.float32),
            pltpu.SemaphoreType.DMA,
            pltpu.SemaphoreType.DMA,
        ],
        compiler_params=pltpu.CompilerParams(use_tc_tiling_on_sc=False),
    )


def _seg(y, src3, dst3):
    return _seg_kernel()(y, src3, dst3)


_BLK_A = 2000


def _stage_a_body(x_ref, wr_ref, wq_ref, t_ref, r_ref):
    x = x_ref[...]
    t_ref[...] = jnp.dot(x, wr_ref[...], preferred_element_type=jnp.float32)
    r_ref[...] = jnp.dot(x, wq_ref[...], preferred_element_type=jnp.float32)


def _stage_a(x, wr, wq):
    grid = (NN // _BLK_A,)
    return pl.pallas_call(
        _stage_a_body,
        grid=grid,
        in_specs=[
            pl.BlockSpec((_BLK_A, D_IN), lambda i: (i, 0)),
            pl.BlockSpec((D_IN, HP), lambda i: (0, 0)),
            pl.BlockSpec((D_IN, HP), lambda i: (0, 0)),
        ],
        out_specs=[pl.BlockSpec((_BLK_A, HP), lambda i: (i, 0))] * 2,
        out_shape=[jax.ShapeDtypeStruct((NN, HP), jnp.float32)] * 2,
    )(x, wr, wq)


_BLK_B = 2000


def _stage_b1_body(p_ref, a_ref, b_ref, wr_ref, h_ref, t_ref):
    h = jnp.maximum(p_ref[0] + p_ref[1] + b_ref[...] + a_ref[...], 0.0)
    h_ref[...] = h
    t_ref[...] = jnp.dot(h, wr_ref[...], preferred_element_type=jnp.float32)


def _stage_b2_body(p_ref, a_ref, b_ref, wq_ref, wr_ref, h_ref, t_ref):
    root = jnp.dot(a_ref[...], wq_ref[...], preferred_element_type=jnp.float32)
    h = jnp.maximum(p_ref[0] + p_ref[1] + b_ref[...] + root, 0.0)
    h_ref[...] = h
    t_ref[...] = jnp.dot(h, wr_ref[...], preferred_element_type=jnp.float32)


def _stage_b3_body(p_ref, a_ref, b_ref, wq_ref, h_ref):
    root = jnp.dot(a_ref[...], wq_ref[...], preferred_element_type=jnp.float32)
    h_ref[...] = jnp.maximum(p_ref[0] + p_ref[1] + b_ref[...] + root, 0.0)


def _stage_b1(p, a, b, wr):
    grid = (NN // _BLK_B,)
    return pl.pallas_call(
        _stage_b1_body,
        grid=grid,
        in_specs=[
            pl.BlockSpec((NC, _BLK_B, HP), lambda i: (0, i, 0)),
            pl.BlockSpec((_BLK_B, HP), lambda i: (i, 0)),
            pl.BlockSpec((1, HP), lambda i: (0, 0)),
            pl.BlockSpec((HP, HP), lambda i: (0, 0)),
        ],
        out_specs=[pl.BlockSpec((_BLK_B, HP), lambda i: (i, 0))] * 2,
        out_shape=[jax.ShapeDtypeStruct((NN, HP), jnp.float32)] * 2,
    )(p, a, b, wr)


def _stage_b2(p, a, b, wq, wr):
    grid = (NN // _BLK_B,)
    return pl.pallas_call(
        _stage_b2_body,
        grid=grid,
        in_specs=[
            pl.BlockSpec((NC, _BLK_B, HP), lambda i: (0, i, 0)),
            pl.BlockSpec((_BLK_B, HP), lambda i: (i, 0)),
            pl.BlockSpec((1, HP), lambda i: (0, 0)),
            pl.BlockSpec((HP, HP), lambda i: (0, 0)),
            pl.BlockSpec((HP, HP), lambda i: (0, 0)),
        ],
        out_specs=[pl.BlockSpec((_BLK_B, HP), lambda i: (i, 0))] * 2,
        out_shape=[jax.ShapeDtypeStruct((NN, HP), jnp.float32)] * 2,
    )(p, a, b, wq, wr)


def _stage_b3(p, a, b, wq):
    grid = (NN // _BLK_B,)
    return pl.pallas_call(
        _stage_b3_body,
        grid=grid,
        in_specs=[
            pl.BlockSpec((NC, _BLK_B, HP), lambda i: (0, i, 0)),
            pl.BlockSpec((_BLK_B, HP), lambda i: (i, 0)),
            pl.BlockSpec((1, HP), lambda i: (0, 0)),
            pl.BlockSpec((HP, HP), lambda i: (0, 0)),
        ],
        out_specs=pl.BlockSpec((_BLK_B, HP), lambda i: (i, 0)),
        out_shape=jax.ShapeDtypeStruct((NN, HP), jnp.float32),
    )(p, a, b, wq)


_BLK_C = 1000
_NBLK_C = NN // _BLK_C
_MOL_BLKS = N // _BLK_C


def _stage_c_body(p_ref, h3_ref, wr_ref, bo_ref, wq_ref, wl1_ref, bl1_ref,
                  wl2_ref, bl2_ref, wh1_ref, bh1_ref, wh2_ref, bh2_ref,
                  wh3_ref, bh3_ref, out_ref, acc):
    i = pl.program_id(0)
    agg = p_ref[0] + p_ref[1]
    h4 = jnp.maximum(
        jnp.dot(agg, wr_ref[...], preferred_element_type=jnp.float32)
        + bo_ref[...]
        + jnp.dot(h3_ref[...], wq_ref[...], preferred_element_type=jnp.float32),
        0.0,
    )
    bs = jnp.sum(h4, axis=0, keepdims=True)

    @pl.when(i == 0)
    def _():
        acc[0:1, :] = bs

    @pl.when((i > 0) & (i < _MOL_BLKS))
    def _():
        acc[0:1, :] = acc[0:1, :] + bs

    @pl.when(i == _MOL_BLKS)
    def _():
        acc[1:2, :] = bs

    @pl.when(i > _MOL_BLKS)
    def _():
        acc[1:2, :] = acc[1:2, :] + bs

    @pl.when(i == _NBLK_C - 1)
    def _():
        m = jnp.maximum(
            jnp.dot(acc[...], wl1_ref[...], preferred_element_type=jnp.float32)
            + bl1_ref[...], 0.0)
        m = jnp.maximum(
            jnp.dot(m, wl2_ref[...], preferred_element_type=jnp.float32)
            + bl2_ref[...], 0.0)
        z = (jnp.dot(m[0:1, :], wh1_ref[0:D_OUT, :],
                     preferred_element_type=jnp.float32)
             + jnp.dot(m[1:2, :], wh1_ref[D_OUT:2 * D_OUT, :],
                       preferred_element_type=jnp.float32)
             + bh1_ref[...])
        z = jnp.maximum(z, 0.0)
        z = jnp.maximum(
            jnp.dot(z, wh2_ref[...], preferred_element_type=jnp.float32)
            + bh2_ref[...], 0.0)
        z = (jnp.dot(z, wh3_ref[...], preferred_element_type=jnp.float32)
             + bh3_ref[...])
        out_ref[...] = 1.0 / (1.0 + jnp.exp(-z))


def _stage_c(p, h3, wr, bo, wq, wl1, bl1, wl2, bl2, wh1, bh1, wh2, bh2, wh3,
             bh3):
    grid = (_NBLK_C,)

    def _full(shape):
        nd = len(shape)
        return pl.BlockSpec(shape, lambda i, _nd=nd: (0,) * _nd)

    return pl.pallas_call(
        _stage_c_body,
        grid=grid,
        in_specs=[
            pl.BlockSpec((NC, _BLK_C, HP), lambda i: (0, i, 0)),
            pl.BlockSpec((_BLK_C, HP), lambda i: (i, 0)),
            _full((HP, D_OUT)),
            _full((1, D_OUT)),
            _full((HP, D_OUT)),
            _full((D_OUT, D_OUT)),
            _full((1, D_OUT)),
            _full((D_OUT, D_OUT)),
            _full((1, D_OUT)),
            _full((2 * D_OUT, 10)),
            _full((1, 10)),
            _full((10, 10)),
            _full((1, 10)),
            _full((10, 1)),
            _full((1, 1)),
        ],
        out_specs=pl.BlockSpec((1, 1), lambda i: (0, 0)),
        out_shape=jax.ShapeDtypeStruct((1, 1), jnp.float32),
        scratch_shapes=[pltpu.VMEM((2, D_OUT), jnp.float32)],
    )(p, h3, wr, bo, wq, wl1, bl1, wl2, bl2, wh1, bh1, wh2, bh2, wh3, bh3)


def _pad_cols(w, width=HP):
    return jnp.pad(w, ((0, 0), (0, width - w.shape[1])))


def _pad_rows(w, height=HP):
    return jnp.pad(w, ((0, height - w.shape[0]), (0, 0)))


def kernel(mol_1_graph, mol_1_nodes, mol_2_graph, mol_2_nodes, params):
    pr = params
    wr_in = _pad_cols(pr['conv_in']['W_rel'])
    wq_in = _pad_cols(pr['conv_in']['W_root'])
    b_in = _pad_cols(pr['conv_in']['b'][None])
    li1, li2 = pr['conv_internal']
    wr1 = _pad_cols(_pad_rows(li1['W_rel']))
    wq1 = _pad_cols(_pad_rows(li1['W_root']))
    b1 = _pad_cols(li1['b'][None])
    wr2 = _pad_cols(_pad_rows(li2['W_rel']))
    wq2 = _pad_cols(_pad_rows(li2['W_root']))
    b2 = _pad_cols(li2['b'][None])
    wr_out = _pad_rows(pr['conv_out']['W_rel'])
    wq_out = _pad_rows(pr['conv_out']['W_root'])
    b_out = pr['conv_out']['b'][None]
    lo1, lo2 = pr['linear_output']
    wh1 = pr['linear_1']['W']
    bh1 = pr['linear_1']['b'][None]
    wh2 = pr['linear_2']['W']
    bh2 = pr['linear_2']['b'][None]
    wh3 = pr['linear_3']['W']
    bh3 = pr['linear_3']['b'][None]

    x = jnp.concatenate([mol_1_nodes, mol_2_nodes], axis=0)
    src = jnp.concatenate([
        mol_1_graph[0], mol_2_graph[0] + N,
        jnp.zeros((EPAD,), jnp.int32),
    ])
    dst = jnp.concatenate([
        mol_1_graph[1], mol_2_graph[1] + N,
        jnp.full((EPAD,), NN, jnp.int32),
    ])
    src3 = jnp.concatenate([
        src.reshape(NW, K, CH),
        jnp.zeros((NW, KP - K, CH), jnp.int32),
    ], axis=1)
    dst3 = jnp.concatenate([
        dst.reshape(NW, K, CH),
        jnp.full((NW, KP - K, CH), NN, jnp.int32),
    ], axis=1)

    t0, r0 = _stage_a(x, wr_in, wq_in)
    p0 = _seg(t0, src3, dst3)
    h1, t1 = _stage_b1(p0, r0, b_in, wr1)
    p1 = _seg(t1, src3, dst3)
    h2, t2 = _stage_b2(p1, h1, b1, wq1, wr2)
    p2 = _seg(t2, src3, dst3)
    h3 = _stage_b3(p2, h2, b2, wq2)
    p3 = _seg(h3, src3, dst3)
    out = _stage_c(p3, h3, wr_out, b_out, wq_out,
                   lo1['W'], lo1['b'][None], lo2['W'], lo2['b'][None],
                   wh1, bh1, wh2, bh2, wh3, bh3)
    return out.reshape((1,))


# speedup vs baseline: 1.9016x; 1.1261x over previous
import functools

import jax
import jax.numpy as jnp
from jax import lax
from jax.experimental import pallas as pl
from jax.experimental.pallas import tpu as pltpu
from jax.experimental.pallas import tpu_sc as plsc

N = 10000
E = 160000
D_IN = 256
HID = 20
HP = 32
D_OUT = 128
NN = 2 * N

NC, NS = 2, 16
NW = NC * NS
CH = 128
E2 = 2 * E
NBUF = 2
K = -(-E2 // (NW * CH * NBUF)) * NBUF
KP = K + NBUF
EP = NW * K * CH
EPAD = EP - E2

ACC_ROWS = 20480
ZROWS = 160
ACC_PER_SUB = ACC_ROWS // NS


def _seg_body(y_hbm, src_hbm, dst_hbm, p_hbm, acc_s, src_v, dst_v, rows_v,
              zbuf_v, sem_g, sem_s):
    c = lax.axis_index("c")
    s = lax.axis_index("s")
    wid = c * NS + s

    pltpu.sync_copy(src_hbm.at[wid], src_v)
    pltpu.sync_copy(dst_hbm.at[wid], dst_v)

    zeros16 = jnp.zeros((16,), jnp.float32)

    def _zfill(i, carry):
        zbuf_v[i // 2, pl.ds((i % 2) * 16, 16)] = zeros16
        return carry

    lax.fori_loop(0, ZROWS * 2, _zfill, 0)
    for t in range(ACC_PER_SUB // ZROWS):
        pltpu.sync_copy(zbuf_v,
                        acc_s.at[pl.ds(s * ACC_PER_SUB + t * ZROWS, ZROWS)])
    plsc.subcore_barrier()

    pltpu.async_copy(y_hbm.at[src_v.at[0]], rows_v.at[0], sem_g)

    def _group(g, carry):
        j0 = g * NBUF
        for b in range(NBUF):
            j = j0 + b
            nb = (b + 1) % NBUF
            pltpu.make_async_copy(y_hbm.at[src_v.at[j]], rows_v.at[b],
                                  sem_g).wait()
            pltpu.async_copy(y_hbm.at[src_v.at[j + 1]], rows_v.at[nb], sem_g)
            pltpu.sync_copy(rows_v.at[b], acc_s.at[dst_v.at[j]], add=True)
        return carry

    lax.fori_loop(0, K // NBUF, _group, 0)
    pltpu.make_async_copy(y_hbm.at[src_v.at[0]], rows_v.at[0], sem_g).wait()
    plsc.subcore_barrier()

    pltpu.sync_copy(acc_s.at[pl.ds(s * ACC_PER_SUB, ACC_PER_SUB)],
                    p_hbm.at[c, pl.ds(s * ACC_PER_SUB, ACC_PER_SUB)])


@functools.lru_cache(maxsize=1)
def _seg_kernel():
    return pl.kernel(
        _seg_body,
        out_type=jax.ShapeDtypeStruct((NC, ACC_ROWS, HP), jnp.float32),
        mesh=plsc.VectorSubcoreMesh(core_axis_name="c", subcore_axis_name="s",
                                    num_cores=NC, num_subcores=NS),
        scratch_types=[
            pltpu.VMEM_SHARED((ACC_ROWS, HP), jnp.float32),
            pltpu.VMEM((KP, CH), jnp.int32),
            pltpu.VMEM((KP, CH), jnp.int32),
            pltpu.VMEM((NBUF, CH, HP), jnp.float32),
            pltpu.VMEM((ZROWS, HP), jnp.float32),
            pltpu.SemaphoreType.DMA,
            pltpu.SemaphoreType.DMA,
        ],
        compiler_params=pltpu.CompilerParams(use_tc_tiling_on_sc=False),
    )


def _seg(y, src3, dst3):
    return _seg_kernel()(y, src3, dst3)


_BLK_A = 2000


def _stage_a_body(x_ref, wr_ref, wq_ref, t_ref, r_ref):
    x = x_ref[...]
    t_ref[...] = jnp.dot(x, wr_ref[...], preferred_element_type=jnp.float32)
    r_ref[...] = jnp.dot(x, wq_ref[...], preferred_element_type=jnp.float32)


def _stage_a(x, wr, wq):
    grid = (NN // _BLK_A,)
    return pl.pallas_call(
        _stage_a_body,
        grid=grid,
        in_specs=[
            pl.BlockSpec((_BLK_A, D_IN), lambda i: (i, 0)),
            pl.BlockSpec((D_IN, HP), lambda i: (0, 0)),
            pl.BlockSpec((D_IN, HP), lambda i: (0, 0)),
        ],
        out_specs=[pl.BlockSpec((_BLK_A, HP), lambda i: (i, 0))] * 2,
        out_shape=[jax.ShapeDtypeStruct((NN, HP), jnp.float32)] * 2,
    )(x, wr, wq)


_BLK_B = 2000


def _stage_b1_body(p_ref, a_ref, b_ref, wr_ref, h_ref, t_ref):
    h = jnp.maximum(p_ref[0] + p_ref[1] + b_ref[...] + a_ref[...], 0.0)
    h_ref[...] = h
    t_ref[...] = jnp.dot(h, wr_ref[...], preferred_element_type=jnp.float32)


def _stage_b2_body(p_ref, a_ref, b_ref, wq_ref, wr_ref, h_ref, t_ref):
    root = jnp.dot(a_ref[...], wq_ref[...], preferred_element_type=jnp.float32)
    h = jnp.maximum(p_ref[0] + p_ref[1] + b_ref[...] + root, 0.0)
    h_ref[...] = h
    t_ref[...] = jnp.dot(h, wr_ref[...], preferred_element_type=jnp.float32)


def _stage_b3_body(p_ref, a_ref, b_ref, wq_ref, h_ref):
    root = jnp.dot(a_ref[...], wq_ref[...], preferred_element_type=jnp.float32)
    h_ref[...] = jnp.maximum(p_ref[0] + p_ref[1] + b_ref[...] + root, 0.0)


def _stage_b1(p, a, b, wr):
    grid = (NN // _BLK_B,)
    return pl.pallas_call(
        _stage_b1_body,
        grid=grid,
        in_specs=[
            pl.BlockSpec((NC, _BLK_B, HP), lambda i: (0, i, 0)),
            pl.BlockSpec((_BLK_B, HP), lambda i: (i, 0)),
            pl.BlockSpec((1, HP), lambda i: (0, 0)),
            pl.BlockSpec((HP, HP), lambda i: (0, 0)),
        ],
        out_specs=[pl.BlockSpec((_BLK_B, HP), lambda i: (i, 0))] * 2,
        out_shape=[jax.ShapeDtypeStruct((NN, HP), jnp.float32)] * 2,
    )(p, a, b, wr)


def _stage_b2(p, a, b, wq, wr):
    grid = (NN // _BLK_B,)
    return pl.pallas_call(
        _stage_b2_body,
        grid=grid,
        in_specs=[
            pl.BlockSpec((NC, _BLK_B, HP), lambda i: (0, i, 0)),
            pl.BlockSpec((_BLK_B, HP), lambda i: (i, 0)),
            pl.BlockSpec((1, HP), lambda i: (0, 0)),
            pl.BlockSpec((HP, HP), lambda i: (0, 0)),
            pl.BlockSpec((HP, HP), lambda i: (0, 0)),
        ],
        out_specs=[pl.BlockSpec((_BLK_B, HP), lambda i: (i, 0))] * 2,
        out_shape=[jax.ShapeDtypeStruct((NN, HP), jnp.float32)] * 2,
    )(p, a, b, wq, wr)


def _stage_b3(p, a, b, wq):
    grid = (NN // _BLK_B,)
    return pl.pallas_call(
        _stage_b3_body,
        grid=grid,
        in_specs=[
            pl.BlockSpec((NC, _BLK_B, HP), lambda i: (0, i, 0)),
            pl.BlockSpec((_BLK_B, HP), lambda i: (i, 0)),
            pl.BlockSpec((1, HP), lambda i: (0, 0)),
            pl.BlockSpec((HP, HP), lambda i: (0, 0)),
        ],
        out_specs=pl.BlockSpec((_BLK_B, HP), lambda i: (i, 0)),
        out_shape=jax.ShapeDtypeStruct((NN, HP), jnp.float32),
    )(p, a, b, wq)


_BLK_C = 1000
_NBLK_C = NN // _BLK_C
_MOL_BLKS = N // _BLK_C


def _stage_c_body(p_ref, h3_ref, wr_ref, bo_ref, wq_ref, wl1_ref, bl1_ref,
                  wl2_ref, bl2_ref, wh1_ref, bh1_ref, wh2_ref, bh2_ref,
                  wh3_ref, bh3_ref, out_ref, acc):
    i = pl.program_id(0)
    agg = p_ref[0] + p_ref[1]
    h4 = jnp.maximum(
        jnp.dot(agg, wr_ref[...], preferred_element_type=jnp.float32)
        + bo_ref[...]
        + jnp.dot(h3_ref[...], wq_ref[...], preferred_element_type=jnp.float32),
        0.0,
    )
    bs = jnp.sum(h4, axis=0, keepdims=True)

    @pl.when(i == 0)
    def _():
        acc[0:1, :] = bs

    @pl.when((i > 0) & (i < _MOL_BLKS))
    def _():
        acc[0:1, :] = acc[0:1, :] + bs

    @pl.when(i == _MOL_BLKS)
    def _():
        acc[1:2, :] = bs

    @pl.when(i > _MOL_BLKS)
    def _():
        acc[1:2, :] = acc[1:2, :] + bs

    @pl.when(i == _NBLK_C - 1)
    def _():
        m = jnp.maximum(
            jnp.dot(acc[...], wl1_ref[...], preferred_element_type=jnp.float32)
            + bl1_ref[...], 0.0)
        m = jnp.maximum(
            jnp.dot(m, wl2_ref[...], preferred_element_type=jnp.float32)
            + bl2_ref[...], 0.0)
        z = (jnp.dot(m[0:1, :], wh1_ref[0:D_OUT, :],
                     preferred_element_type=jnp.float32)
             + jnp.dot(m[1:2, :], wh1_ref[D_OUT:2 * D_OUT, :],
                       preferred_element_type=jnp.float32)
             + bh1_ref[...])
        z = jnp.maximum(z, 0.0)
        z = jnp.maximum(
            jnp.dot(z, wh2_ref[...], preferred_element_type=jnp.float32)
            + bh2_ref[...], 0.0)
        z = (jnp.dot(z, wh3_ref[...], preferred_element_type=jnp.float32)
             + bh3_ref[...])
        out_ref[...] = 1.0 / (1.0 + jnp.exp(-z))


def _stage_c(p, h3, wr, bo, wq, wl1, bl1, wl2, bl2, wh1, bh1, wh2, bh2, wh3,
             bh3):
    grid = (_NBLK_C,)

    def _full(shape):
        nd = len(shape)
        return pl.BlockSpec(shape, lambda i, _nd=nd: (0,) * _nd)

    return pl.pallas_call(
        _stage_c_body,
        grid=grid,
        in_specs=[
            pl.BlockSpec((NC, _BLK_C, HP), lambda i: (0, i, 0)),
            pl.BlockSpec((_BLK_C, HP), lambda i: (i, 0)),
            _full((HP, D_OUT)),
            _full((1, D_OUT)),
            _full((HP, D_OUT)),
            _full((D_OUT, D_OUT)),
            _full((1, D_OUT)),
            _full((D_OUT, D_OUT)),
            _full((1, D_OUT)),
            _full((2 * D_OUT, 10)),
            _full((1, 10)),
            _full((10, 10)),
            _full((1, 10)),
            _full((10, 1)),
            _full((1, 1)),
        ],
        out_specs=pl.BlockSpec((1, 1), lambda i: (0, 0)),
        out_shape=jax.ShapeDtypeStruct((1, 1), jnp.float32),
        scratch_shapes=[pltpu.VMEM((2, D_OUT), jnp.float32)],
    )(p, h3, wr, bo, wq, wl1, bl1, wl2, bl2, wh1, bh1, wh2, bh2, wh3, bh3)


def _pad_cols(w, width=HP):
    return jnp.pad(w, ((0, 0), (0, width - w.shape[1])))


def _pad_rows(w, height=HP):
    return jnp.pad(w, ((0, height - w.shape[0]), (0, 0)))


def kernel(mol_1_graph, mol_1_nodes, mol_2_graph, mol_2_nodes, params):
    pr = params
    wr_in = _pad_cols(pr['conv_in']['W_rel'])
    wq_in = _pad_cols(pr['conv_in']['W_root'])
    b_in = _pad_cols(pr['conv_in']['b'][None])
    li1, li2 = pr['conv_internal']
    wr1 = _pad_cols(_pad_rows(li1['W_rel']))
    wq1 = _pad_cols(_pad_rows(li1['W_root']))
    b1 = _pad_cols(li1['b'][None])
    wr2 = _pad_cols(_pad_rows(li2['W_rel']))
    wq2 = _pad_cols(_pad_rows(li2['W_root']))
    b2 = _pad_cols(li2['b'][None])
    wr_out = _pad_rows(pr['conv_out']['W_rel'])
    wq_out = _pad_rows(pr['conv_out']['W_root'])
    b_out = pr['conv_out']['b'][None]
    lo1, lo2 = pr['linear_output']
    wh1 = pr['linear_1']['W']
    bh1 = pr['linear_1']['b'][None]
    wh2 = pr['linear_2']['W']
    bh2 = pr['linear_2']['b'][None]
    wh3 = pr['linear_3']['W']
    bh3 = pr['linear_3']['b'][None]

    x = jnp.concatenate([mol_1_nodes, mol_2_nodes], axis=0)
    src = jnp.concatenate([
        mol_1_graph[0], mol_2_graph[0] + N,
        jnp.zeros((EPAD,), jnp.int32),
    ])
    dst = jnp.concatenate([
        mol_1_graph[1], mol_2_graph[1] + N,
        jnp.full((EPAD,), NN, jnp.int32),
    ])
    src3 = jnp.concatenate([
        src.reshape(NW, K, CH),
        jnp.zeros((NW, KP - K, CH), jnp.int32),
    ], axis=1)
    dst3 = jnp.concatenate([
        dst.reshape(NW, K, CH),
        jnp.full((NW, KP - K, CH), NN, jnp.int32),
    ], axis=1)

    t0, r0 = _stage_a(x, wr_in, wq_in)
    p0 = _seg(t0, src3, dst3)
    h1, t1 = _stage_b1(p0, r0, b_in, wr1)
    p1 = _seg(t1, src3, dst3)
    h2, t2 = _stage_b2(p1, h1, b1, wq1, wr2)
    p2 = _seg(t2, src3, dst3)
    h3 = _stage_b3(p2, h2, b2, wq2)
    p3 = _seg(h3, src3, dst3)
    out = _stage_c(p3, h3, wr_out, b_out, wq_out,
                   lo1['W'], lo1['b'][None], lo2['W'], lo2['b'][None],
                   wh1, bh1, wh2, bh2, wh3, bh3)
    return out.reshape((1,))


# speedup vs baseline: 4.3624x; 2.2941x over previous
import functools

import jax
import jax.numpy as jnp
from jax import lax
from jax.experimental import pallas as pl
from jax.experimental.pallas import tpu as pltpu
from jax.experimental.pallas import tpu_sc as plsc

N = 10000
E = 160000
D_IN = 256
HID = 20
HP = 32
D_OUT = 128
NN = 2 * N

NC, NS = 2, 16
NW = NC * NS
CH = 128
E2 = 2 * E
K = -(-E2 // (NW * CH))
KP = K + 1
EP = NW * K * CH
EPAD = EP - E2
YSTG = NN // NS

ACC_ROWS = 20480
ZROWS = 160
ACC_PER_SUB = ACC_ROWS // NS


def _seg_body(y_hbm, src_hbm, dst_hbm, p_hbm, acc_s, y_s, src_v, dst_v,
              rows_v, zbuf_v, sem_g, sem_s):
    c = lax.axis_index("c")
    s = lax.axis_index("s")
    wid = c * NS + s

    pltpu.sync_copy(src_hbm.at[wid], src_v)
    pltpu.sync_copy(dst_hbm.at[wid], dst_v)
    pltpu.sync_copy(y_hbm.at[pl.ds(s * YSTG, YSTG)],
                    y_s.at[pl.ds(s * YSTG, YSTG)])

    zeros16 = jnp.zeros((16,), jnp.float32)

    def _zfill(i, carry):
        zbuf_v[i // 2, pl.ds((i % 2) * 16, 16)] = zeros16
        return carry

    lax.fori_loop(0, ZROWS * 2, _zfill, 0)
    for t in range(ACC_PER_SUB // ZROWS):
        pltpu.sync_copy(zbuf_v,
                        acc_s.at[pl.ds(s * ACC_PER_SUB + t * ZROWS, ZROWS)])
    plsc.subcore_barrier()

    def _chunk(j, carry):
        pltpu.async_copy(y_s.at[src_v.at[j]], rows_v, sem_g).wait()
        pltpu.sync_copy(rows_v, acc_s.at[dst_v.at[j]], add=True)
        return carry

    lax.fori_loop(0, K, _chunk, 0)
    plsc.subcore_barrier()

    pltpu.sync_copy(acc_s.at[pl.ds(s * ACC_PER_SUB, ACC_PER_SUB)],
                    p_hbm.at[c, pl.ds(s * ACC_PER_SUB, ACC_PER_SUB)])


@functools.lru_cache(maxsize=1)
def _seg_kernel():
    return pl.kernel(
        _seg_body,
        out_type=jax.ShapeDtypeStruct((NC, ACC_ROWS, HP), jnp.float32),
        mesh=plsc.VectorSubcoreMesh(core_axis_name="c", subcore_axis_name="s",
                                    num_cores=NC, num_subcores=NS),
        scratch_types=[
            pltpu.VMEM_SHARED((ACC_ROWS, HP), jnp.float32),
            pltpu.VMEM_SHARED((NN, HP), jnp.float32),
            pltpu.VMEM((KP, CH), jnp.int32),
            pltpu.VMEM((KP, CH), jnp.int32),
            pltpu.VMEM((CH, HP), jnp.float32),
            pltpu.VMEM((ZROWS, HP), jnp.float32),
            pltpu.SemaphoreType.DMA,
            pltpu.SemaphoreType.DMA,
        ],
        compiler_params=pltpu.CompilerParams(use_tc_tiling_on_sc=False),
    )


def _seg(y, src3, dst3):
    return _seg_kernel()(y, src3, dst3)


_BLK_A = 2000


def _stage_a_body(x_ref, wr_ref, wq_ref, t_ref, r_ref):
    x = x_ref[...]
    t_ref[...] = jnp.dot(x, wr_ref[...], preferred_element_type=jnp.float32)
    r_ref[...] = jnp.dot(x, wq_ref[...], preferred_element_type=jnp.float32)


def _stage_a(x, wr, wq):
    grid = (NN // _BLK_A,)
    return pl.pallas_call(
        _stage_a_body,
        grid=grid,
        in_specs=[
            pl.BlockSpec((_BLK_A, D_IN), lambda i: (i, 0)),
            pl.BlockSpec((D_IN, HP), lambda i: (0, 0)),
            pl.BlockSpec((D_IN, HP), lambda i: (0, 0)),
        ],
        out_specs=[pl.BlockSpec((_BLK_A, HP), lambda i: (i, 0))] * 2,
        out_shape=[jax.ShapeDtypeStruct((NN, HP), jnp.float32)] * 2,
    )(x, wr, wq)


_BLK_B = 2000


def _stage_b1_body(p_ref, a_ref, b_ref, wr_ref, h_ref, t_ref):
    h = jnp.maximum(p_ref[0] + p_ref[1] + b_ref[...] + a_ref[...], 0.0)
    h_ref[...] = h
    t_ref[...] = jnp.dot(h, wr_ref[...], preferred_element_type=jnp.float32)


def _stage_b2_body(p_ref, a_ref, b_ref, wq_ref, wr_ref, h_ref, t_ref):
    root = jnp.dot(a_ref[...], wq_ref[...], preferred_element_type=jnp.float32)
    h = jnp.maximum(p_ref[0] + p_ref[1] + b_ref[...] + root, 0.0)
    h_ref[...] = h
    t_ref[...] = jnp.dot(h, wr_ref[...], preferred_element_type=jnp.float32)


def _stage_b3_body(p_ref, a_ref, b_ref, wq_ref, h_ref):
    root = jnp.dot(a_ref[...], wq_ref[...], preferred_element_type=jnp.float32)
    h_ref[...] = jnp.maximum(p_ref[0] + p_ref[1] + b_ref[...] + root, 0.0)


def _stage_b1(p, a, b, wr):
    grid = (NN // _BLK_B,)
    return pl.pallas_call(
        _stage_b1_body,
        grid=grid,
        in_specs=[
            pl.BlockSpec((NC, _BLK_B, HP), lambda i: (0, i, 0)),
            pl.BlockSpec((_BLK_B, HP), lambda i: (i, 0)),
            pl.BlockSpec((1, HP), lambda i: (0, 0)),
            pl.BlockSpec((HP, HP), lambda i: (0, 0)),
        ],
        out_specs=[pl.BlockSpec((_BLK_B, HP), lambda i: (i, 0))] * 2,
        out_shape=[jax.ShapeDtypeStruct((NN, HP), jnp.float32)] * 2,
    )(p, a, b, wr)


def _stage_b2(p, a, b, wq, wr):
    grid = (NN // _BLK_B,)
    return pl.pallas_call(
        _stage_b2_body,
        grid=grid,
        in_specs=[
            pl.BlockSpec((NC, _BLK_B, HP), lambda i: (0, i, 0)),
            pl.BlockSpec((_BLK_B, HP), lambda i: (i, 0)),
            pl.BlockSpec((1, HP), lambda i: (0, 0)),
            pl.BlockSpec((HP, HP), lambda i: (0, 0)),
            pl.BlockSpec((HP, HP), lambda i: (0, 0)),
        ],
        out_specs=[pl.BlockSpec((_BLK_B, HP), lambda i: (i, 0))] * 2,
        out_shape=[jax.ShapeDtypeStruct((NN, HP), jnp.float32)] * 2,
    )(p, a, b, wq, wr)


def _stage_b3(p, a, b, wq):
    grid = (NN // _BLK_B,)
    return pl.pallas_call(
        _stage_b3_body,
        grid=grid,
        in_specs=[
            pl.BlockSpec((NC, _BLK_B, HP), lambda i: (0, i, 0)),
            pl.BlockSpec((_BLK_B, HP), lambda i: (i, 0)),
            pl.BlockSpec((1, HP), lambda i: (0, 0)),
            pl.BlockSpec((HP, HP), lambda i: (0, 0)),
        ],
        out_specs=pl.BlockSpec((_BLK_B, HP), lambda i: (i, 0)),
        out_shape=jax.ShapeDtypeStruct((NN, HP), jnp.float32),
    )(p, a, b, wq)


_BLK_C = 1000
_NBLK_C = NN // _BLK_C
_MOL_BLKS = N // _BLK_C


def _stage_c_body(p_ref, h3_ref, wr_ref, bo_ref, wq_ref, wl1_ref, bl1_ref,
                  wl2_ref, bl2_ref, wh1_ref, bh1_ref, wh2_ref, bh2_ref,
                  wh3_ref, bh3_ref, out_ref, acc):
    i = pl.program_id(0)
    agg = p_ref[0] + p_ref[1]
    h4 = jnp.maximum(
        jnp.dot(agg, wr_ref[...], preferred_element_type=jnp.float32)
        + bo_ref[...]
        + jnp.dot(h3_ref[...], wq_ref[...], preferred_element_type=jnp.float32),
        0.0,
    )
    bs = jnp.sum(h4, axis=0, keepdims=True)

    @pl.when(i == 0)
    def _():
        acc[0:1, :] = bs

    @pl.when((i > 0) & (i < _MOL_BLKS))
    def _():
        acc[0:1, :] = acc[0:1, :] + bs

    @pl.when(i == _MOL_BLKS)
    def _():
        acc[1:2, :] = bs

    @pl.when(i > _MOL_BLKS)
    def _():
        acc[1:2, :] = acc[1:2, :] + bs

    @pl.when(i == _NBLK_C - 1)
    def _():
        m = jnp.maximum(
            jnp.dot(acc[...], wl1_ref[...], preferred_element_type=jnp.float32)
            + bl1_ref[...], 0.0)
        m = jnp.maximum(
            jnp.dot(m, wl2_ref[...], preferred_element_type=jnp.float32)
            + bl2_ref[...], 0.0)
        z = (jnp.dot(m[0:1, :], wh1_ref[0:D_OUT, :],
                     preferred_element_type=jnp.float32)
             + jnp.dot(m[1:2, :], wh1_ref[D_OUT:2 * D_OUT, :],
                       preferred_element_type=jnp.float32)
             + bh1_ref[...])
        z = jnp.maximum(z, 0.0)
        z = jnp.maximum(
            jnp.dot(z, wh2_ref[...], preferred_element_type=jnp.float32)
            + bh2_ref[...], 0.0)
        z = (jnp.dot(z, wh3_ref[...], preferred_element_type=jnp.float32)
             + bh3_ref[...])
        out_ref[...] = 1.0 / (1.0 + jnp.exp(-z))


def _stage_c(p, h3, wr, bo, wq, wl1, bl1, wl2, bl2, wh1, bh1, wh2, bh2, wh3,
             bh3):
    grid = (_NBLK_C,)

    def _full(shape):
        nd = len(shape)
        return pl.BlockSpec(shape, lambda i, _nd=nd: (0,) * _nd)

    return pl.pallas_call(
        _stage_c_body,
        grid=grid,
        in_specs=[
            pl.BlockSpec((NC, _BLK_C, HP), lambda i: (0, i, 0)),
            pl.BlockSpec((_BLK_C, HP), lambda i: (i, 0)),
            _full((HP, D_OUT)),
            _full((1, D_OUT)),
            _full((HP, D_OUT)),
            _full((D_OUT, D_OUT)),
            _full((1, D_OUT)),
            _full((D_OUT, D_OUT)),
            _full((1, D_OUT)),
            _full((2 * D_OUT, 10)),
            _full((1, 10)),
            _full((10, 10)),
            _full((1, 10)),
            _full((10, 1)),
            _full((1, 1)),
        ],
        out_specs=pl.BlockSpec((1, 1), lambda i: (0, 0)),
        out_shape=jax.ShapeDtypeStruct((1, 1), jnp.float32),
        scratch_shapes=[pltpu.VMEM((2, D_OUT), jnp.float32)],
    )(p, h3, wr, bo, wq, wl1, bl1, wl2, bl2, wh1, bh1, wh2, bh2, wh3, bh3)


def _pad_cols(w, width=HP):
    return jnp.pad(w, ((0, 0), (0, width - w.shape[1])))


def _pad_rows(w, height=HP):
    return jnp.pad(w, ((0, height - w.shape[0]), (0, 0)))


def kernel(mol_1_graph, mol_1_nodes, mol_2_graph, mol_2_nodes, params):
    pr = params
    wr_in = _pad_cols(pr['conv_in']['W_rel'])
    wq_in = _pad_cols(pr['conv_in']['W_root'])
    b_in = _pad_cols(pr['conv_in']['b'][None])
    li1, li2 = pr['conv_internal']
    wr1 = _pad_cols(_pad_rows(li1['W_rel']))
    wq1 = _pad_cols(_pad_rows(li1['W_root']))
    b1 = _pad_cols(li1['b'][None])
    wr2 = _pad_cols(_pad_rows(li2['W_rel']))
    wq2 = _pad_cols(_pad_rows(li2['W_root']))
    b2 = _pad_cols(li2['b'][None])
    wr_out = _pad_rows(pr['conv_out']['W_rel'])
    wq_out = _pad_rows(pr['conv_out']['W_root'])
    b_out = pr['conv_out']['b'][None]
    lo1, lo2 = pr['linear_output']
    wh1 = pr['linear_1']['W']
    bh1 = pr['linear_1']['b'][None]
    wh2 = pr['linear_2']['W']
    bh2 = pr['linear_2']['b'][None]
    wh3 = pr['linear_3']['W']
    bh3 = pr['linear_3']['b'][None]

    x = jnp.concatenate([mol_1_nodes, mol_2_nodes], axis=0)
    src = jnp.concatenate([
        mol_1_graph[0], mol_2_graph[0] + N,
        jnp.zeros((EPAD,), jnp.int32),
    ])
    dst = jnp.concatenate([
        mol_1_graph[1], mol_2_graph[1] + N,
        jnp.full((EPAD,), NN, jnp.int32),
    ])
    src3 = jnp.concatenate([
        src.reshape(NW, K, CH),
        jnp.zeros((NW, KP - K, CH), jnp.int32),
    ], axis=1)
    dst3 = jnp.concatenate([
        dst.reshape(NW, K, CH),
        jnp.full((NW, KP - K, CH), NN, jnp.int32),
    ], axis=1)

    t0, r0 = _stage_a(x, wr_in, wq_in)
    p0 = _seg(t0, src3, dst3)
    h1, t1 = _stage_b1(p0, r0, b_in, wr1)
    p1 = _seg(t1, src3, dst3)
    h2, t2 = _stage_b2(p1, h1, b1, wq1, wr2)
    p2 = _seg(t2, src3, dst3)
    h3 = _stage_b3(p2, h2, b2, wq2)
    p3 = _seg(h3, src3, dst3)
    out = _stage_c(p3, h3, wr_out, b_out, wq_out,
                   lo1['W'], lo1['b'][None], lo2['W'], lo2['b'][None],
                   wh1, bh1, wh2, bh2, wh3, bh3)
    return out.reshape((1,))


# speedup vs baseline: 4.8391x; 1.1093x over previous
import functools

import jax
import jax.numpy as jnp
from jax import lax
from jax.experimental import pallas as pl
from jax.experimental.pallas import tpu as pltpu
from jax.experimental.pallas import tpu_sc as plsc

N = 10000
E = 160000
D_IN = 256
HID = 20
HP = 32
D_OUT = 128
NN = 2 * N

NC, NS = 2, 16
NW = NC * NS
CH = 128
E2 = 2 * E
K = -(-E2 // (NW * CH))
KP = K + 1
EP = NW * K * CH
EPAD = EP - E2
YSTG = NN // NS

ACC_ROWS = 20480
ZROWS = 160
ACC_PER_SUB = ACC_ROWS // NS


def _seg_body(y_hbm, src_hbm, dst_hbm, p_hbm, acc_s, y_s, src_v, dst_v,
              rows_v, zbuf_v, sem_g, sem_s):
    c = lax.axis_index("c")
    s = lax.axis_index("s")
    wid = c * NS + s

    pltpu.sync_copy(src_hbm.at[wid], src_v)
    pltpu.sync_copy(dst_hbm.at[wid], dst_v)
    pltpu.sync_copy(y_hbm.at[pl.ds(s * YSTG, YSTG)],
                    y_s.at[pl.ds(s * YSTG, YSTG)])

    zeros16 = jnp.zeros((16,), jnp.float32)

    def _zfill(i, carry):
        zbuf_v[i // 2, pl.ds((i % 2) * 16, 16)] = zeros16
        return carry

    lax.fori_loop(0, ZROWS * 2, _zfill, 0)
    for t in range(ACC_PER_SUB // ZROWS):
        pltpu.sync_copy(zbuf_v,
                        acc_s.at[pl.ds(s * ACC_PER_SUB + t * ZROWS, ZROWS)])
    plsc.subcore_barrier()

    pltpu.async_copy(y_s.at[src_v.at[0]], rows_v.at[0], sem_g)

    def _pair(g, carry):
        for b in range(2):
            j = 2 * g + b
            pltpu.make_async_copy(y_s.at[src_v.at[j]], rows_v.at[b],
                                  sem_g).wait()
            pltpu.async_copy(y_s.at[src_v.at[j + 1]], rows_v.at[1 - b], sem_g)
            pltpu.sync_copy(rows_v.at[b], acc_s.at[dst_v.at[j]], add=True)
        return carry

    lax.fori_loop(0, K // 2, _pair, 0)
    pltpu.make_async_copy(y_s.at[src_v.at[K - 1]], rows_v.at[0], sem_g).wait()
    pltpu.async_copy(y_s.at[src_v.at[K]], rows_v.at[1], sem_g)
    pltpu.sync_copy(rows_v.at[0], acc_s.at[dst_v.at[K - 1]], add=True)
    pltpu.make_async_copy(y_s.at[src_v.at[K]], rows_v.at[1], sem_g).wait()
    plsc.subcore_barrier()

    pltpu.sync_copy(acc_s.at[pl.ds(s * ACC_PER_SUB, ACC_PER_SUB)],
                    p_hbm.at[c, pl.ds(s * ACC_PER_SUB, ACC_PER_SUB)])


@functools.lru_cache(maxsize=1)
def _seg_kernel():
    return pl.kernel(
        _seg_body,
        out_type=jax.ShapeDtypeStruct((NC, ACC_ROWS, HP), jnp.float32),
        mesh=plsc.VectorSubcoreMesh(core_axis_name="c", subcore_axis_name="s",
                                    num_cores=NC, num_subcores=NS),
        scratch_types=[
            pltpu.VMEM_SHARED((ACC_ROWS, HP), jnp.float32),
            pltpu.VMEM_SHARED((NN, HP), jnp.float32),
            pltpu.VMEM((KP, CH), jnp.int32),
            pltpu.VMEM((KP, CH), jnp.int32),
            pltpu.VMEM((2, CH, HP), jnp.float32),
            pltpu.VMEM((ZROWS, HP), jnp.float32),
            pltpu.SemaphoreType.DMA,
            pltpu.SemaphoreType.DMA,
        ],
        compiler_params=pltpu.CompilerParams(use_tc_tiling_on_sc=False),
    )


def _seg(y, src3, dst3):
    return _seg_kernel()(y, src3, dst3)


_BLK_A = 2000


def _stage_a_body(x_ref, wr_ref, wq_ref, t_ref, r_ref):
    x = x_ref[...]
    t_ref[...] = jnp.dot(x, wr_ref[...], preferred_element_type=jnp.float32)
    r_ref[...] = jnp.dot(x, wq_ref[...], preferred_element_type=jnp.float32)


def _stage_a(x, wr, wq):
    grid = (NN // _BLK_A,)
    return pl.pallas_call(
        _stage_a_body,
        grid=grid,
        in_specs=[
            pl.BlockSpec((_BLK_A, D_IN), lambda i: (i, 0)),
            pl.BlockSpec((D_IN, HP), lambda i: (0, 0)),
            pl.BlockSpec((D_IN, HP), lambda i: (0, 0)),
        ],
        out_specs=[pl.BlockSpec((_BLK_A, HP), lambda i: (i, 0))] * 2,
        out_shape=[jax.ShapeDtypeStruct((NN, HP), jnp.float32)] * 2,
    )(x, wr, wq)


_BLK_B = 2000


def _stage_b1_body(p_ref, a_ref, b_ref, wr_ref, h_ref, t_ref):
    h = jnp.maximum(p_ref[0] + p_ref[1] + b_ref[...] + a_ref[...], 0.0)
    h_ref[...] = h
    t_ref[...] = jnp.dot(h, wr_ref[...], preferred_element_type=jnp.float32)


def _stage_b2_body(p_ref, a_ref, b_ref, wq_ref, wr_ref, h_ref, t_ref):
    root = jnp.dot(a_ref[...], wq_ref[...], preferred_element_type=jnp.float32)
    h = jnp.maximum(p_ref[0] + p_ref[1] + b_ref[...] + root, 0.0)
    h_ref[...] = h
    t_ref[...] = jnp.dot(h, wr_ref[...], preferred_element_type=jnp.float32)


def _stage_b3_body(p_ref, a_ref, b_ref, wq_ref, h_ref):
    root = jnp.dot(a_ref[...], wq_ref[...], preferred_element_type=jnp.float32)
    h_ref[...] = jnp.maximum(p_ref[0] + p_ref[1] + b_ref[...] + root, 0.0)


def _stage_b1(p, a, b, wr):
    grid = (NN // _BLK_B,)
    return pl.pallas_call(
        _stage_b1_body,
        grid=grid,
        in_specs=[
            pl.BlockSpec((NC, _BLK_B, HP), lambda i: (0, i, 0)),
            pl.BlockSpec((_BLK_B, HP), lambda i: (i, 0)),
            pl.BlockSpec((1, HP), lambda i: (0, 0)),
            pl.BlockSpec((HP, HP), lambda i: (0, 0)),
        ],
        out_specs=[pl.BlockSpec((_BLK_B, HP), lambda i: (i, 0))] * 2,
        out_shape=[jax.ShapeDtypeStruct((NN, HP), jnp.float32)] * 2,
    )(p, a, b, wr)


def _stage_b2(p, a, b, wq, wr):
    grid = (NN // _BLK_B,)
    return pl.pallas_call(
        _stage_b2_body,
        grid=grid,
        in_specs=[
            pl.BlockSpec((NC, _BLK_B, HP), lambda i: (0, i, 0)),
            pl.BlockSpec((_BLK_B, HP), lambda i: (i, 0)),
            pl.BlockSpec((1, HP), lambda i: (0, 0)),
            pl.BlockSpec((HP, HP), lambda i: (0, 0)),
            pl.BlockSpec((HP, HP), lambda i: (0, 0)),
        ],
        out_specs=[pl.BlockSpec((_BLK_B, HP), lambda i: (i, 0))] * 2,
        out_shape=[jax.ShapeDtypeStruct((NN, HP), jnp.float32)] * 2,
    )(p, a, b, wq, wr)


def _stage_b3(p, a, b, wq):
    grid = (NN // _BLK_B,)
    return pl.pallas_call(
        _stage_b3_body,
        grid=grid,
        in_specs=[
            pl.BlockSpec((NC, _BLK_B, HP), lambda i: (0, i, 0)),
            pl.BlockSpec((_BLK_B, HP), lambda i: (i, 0)),
            pl.BlockSpec((1, HP), lambda i: (0, 0)),
            pl.BlockSpec((HP, HP), lambda i: (0, 0)),
        ],
        out_specs=pl.BlockSpec((_BLK_B, HP), lambda i: (i, 0)),
        out_shape=jax.ShapeDtypeStruct((NN, HP), jnp.float32),
    )(p, a, b, wq)


_BLK_C = 1000
_NBLK_C = NN // _BLK_C
_MOL_BLKS = N // _BLK_C


def _stage_c_body(p_ref, h3_ref, wr_ref, bo_ref, wq_ref, wl1_ref, bl1_ref,
                  wl2_ref, bl2_ref, wh1_ref, bh1_ref, wh2_ref, bh2_ref,
                  wh3_ref, bh3_ref, out_ref, acc):
    i = pl.program_id(0)
    agg = p_ref[0] + p_ref[1]
    h4 = jnp.maximum(
        jnp.dot(agg, wr_ref[...], preferred_element_type=jnp.float32)
        + bo_ref[...]
        + jnp.dot(h3_ref[...], wq_ref[...], preferred_element_type=jnp.float32),
        0.0,
    )
    bs = jnp.sum(h4, axis=0, keepdims=True)

    @pl.when(i == 0)
    def _():
        acc[0:1, :] = bs

    @pl.when((i > 0) & (i < _MOL_BLKS))
    def _():
        acc[0:1, :] = acc[0:1, :] + bs

    @pl.when(i == _MOL_BLKS)
    def _():
        acc[1:2, :] = bs

    @pl.when(i > _MOL_BLKS)
    def _():
        acc[1:2, :] = acc[1:2, :] + bs

    @pl.when(i == _NBLK_C - 1)
    def _():
        m = jnp.maximum(
            jnp.dot(acc[...], wl1_ref[...], preferred_element_type=jnp.float32)
            + bl1_ref[...], 0.0)
        m = jnp.maximum(
            jnp.dot(m, wl2_ref[...], preferred_element_type=jnp.float32)
            + bl2_ref[...], 0.0)
        z = (jnp.dot(m[0:1, :], wh1_ref[0:D_OUT, :],
                     preferred_element_type=jnp.float32)
             + jnp.dot(m[1:2, :], wh1_ref[D_OUT:2 * D_OUT, :],
                       preferred_element_type=jnp.float32)
             + bh1_ref[...])
        z = jnp.maximum(z, 0.0)
        z = jnp.maximum(
            jnp.dot(z, wh2_ref[...], preferred_element_type=jnp.float32)
            + bh2_ref[...], 0.0)
        z = (jnp.dot(z, wh3_ref[...], preferred_element_type=jnp.float32)
             + bh3_ref[...])
        out_ref[...] = 1.0 / (1.0 + jnp.exp(-z))


def _stage_c(p, h3, wr, bo, wq, wl1, bl1, wl2, bl2, wh1, bh1, wh2, bh2, wh3,
             bh3):
    grid = (_NBLK_C,)

    def _full(shape):
        nd = len(shape)
        return pl.BlockSpec(shape, lambda i, _nd=nd: (0,) * _nd)

    return pl.pallas_call(
        _stage_c_body,
        grid=grid,
        in_specs=[
            pl.BlockSpec((NC, _BLK_C, HP), lambda i: (0, i, 0)),
            pl.BlockSpec((_BLK_C, HP), lambda i: (i, 0)),
            _full((HP, D_OUT)),
            _full((1, D_OUT)),
            _full((HP, D_OUT)),
            _full((D_OUT, D_OUT)),
            _full((1, D_OUT)),
            _full((D_OUT, D_OUT)),
            _full((1, D_OUT)),
            _full((2 * D_OUT, 10)),
            _full((1, 10)),
            _full((10, 10)),
            _full((1, 10)),
            _full((10, 1)),
            _full((1, 1)),
        ],
        out_specs=pl.BlockSpec((1, 1), lambda i: (0, 0)),
        out_shape=jax.ShapeDtypeStruct((1, 1), jnp.float32),
        scratch_shapes=[pltpu.VMEM((2, D_OUT), jnp.float32)],
    )(p, h3, wr, bo, wq, wl1, bl1, wl2, bl2, wh1, bh1, wh2, bh2, wh3, bh3)


def _pad_cols(w, width=HP):
    return jnp.pad(w, ((0, 0), (0, width - w.shape[1])))


def _pad_rows(w, height=HP):
    return jnp.pad(w, ((0, height - w.shape[0]), (0, 0)))


def kernel(mol_1_graph, mol_1_nodes, mol_2_graph, mol_2_nodes, params):
    pr = params
    wr_in = _pad_cols(pr['conv_in']['W_rel'])
    wq_in = _pad_cols(pr['conv_in']['W_root'])
    b_in = _pad_cols(pr['conv_in']['b'][None])
    li1, li2 = pr['conv_internal']
    wr1 = _pad_cols(_pad_rows(li1['W_rel']))
    wq1 = _pad_cols(_pad_rows(li1['W_root']))
    b1 = _pad_cols(li1['b'][None])
    wr2 = _pad_cols(_pad_rows(li2['W_rel']))
    wq2 = _pad_cols(_pad_rows(li2['W_root']))
    b2 = _pad_cols(li2['b'][None])
    wr_out = _pad_rows(pr['conv_out']['W_rel'])
    wq_out = _pad_rows(pr['conv_out']['W_root'])
    b_out = pr['conv_out']['b'][None]
    lo1, lo2 = pr['linear_output']
    wh1 = pr['linear_1']['W']
    bh1 = pr['linear_1']['b'][None]
    wh2 = pr['linear_2']['W']
    bh2 = pr['linear_2']['b'][None]
    wh3 = pr['linear_3']['W']
    bh3 = pr['linear_3']['b'][None]

    x = jnp.concatenate([mol_1_nodes, mol_2_nodes], axis=0)
    src = jnp.concatenate([
        mol_1_graph[0], mol_2_graph[0] + N,
        jnp.zeros((EPAD,), jnp.int32),
    ])
    dst = jnp.concatenate([
        mol_1_graph[1], mol_2_graph[1] + N,
        jnp.full((EPAD,), NN, jnp.int32),
    ])
    src3 = jnp.concatenate([
        src.reshape(NW, K, CH),
        jnp.zeros((NW, KP - K, CH), jnp.int32),
    ], axis=1)
    dst3 = jnp.concatenate([
        dst.reshape(NW, K, CH),
        jnp.full((NW, KP - K, CH), NN, jnp.int32),
    ], axis=1)

    t0, r0 = _stage_a(x, wr_in, wq_in)
    p0 = _seg(t0, src3, dst3)
    h1, t1 = _stage_b1(p0, r0, b_in, wr1)
    p1 = _seg(t1, src3, dst3)
    h2, t2 = _stage_b2(p1, h1, b1, wq1, wr2)
    p2 = _seg(t2, src3, dst3)
    h3 = _stage_b3(p2, h2, b2, wq2)
    p3 = _seg(h3, src3, dst3)
    out = _stage_c(p3, h3, wr_out, b_out, wq_out,
                   lo1['W'], lo1['b'][None], lo2['W'], lo2['b'][None],
                   wh1, bh1, wh2, bh2, wh3, bh3)
    return out.reshape((1,))


# speedup vs baseline: 6.3400x; 1.3102x over previous
import functools

import jax
import jax.numpy as jnp
import numpy as np
from jax import lax
from jax.experimental import pallas as pl
from jax.experimental.pallas import tpu as pltpu
from jax.experimental.pallas import tpu_sc as plsc

N = 10000
E = 160000
D_IN = 256
HID = 20
HP = 32
D_OUT = 128
NN = 2 * N

NC, NS = 2, 16
NW = NC * NS
CH = 128
E2 = 2 * E
K = -(-E2 // (NW * CH))
KP = K + 1
EP = NW * K * CH
EPAD = EP - E2

ACC_ROWS = 20480
ZROWS = 160
ACC_PER_SUB = ACC_ROWS // NS
YSTG = NN // NS

KS = ACC_ROWS // (NW * CH)
G = ACC_ROWS * HP // 1024
GB = 64
GG = 625

_SELF_IDS = np.arange(ACC_ROWS, dtype=np.int32)
_SELF_SRC3 = np.where(_SELF_IDS < NN, _SELF_IDS, 0).reshape(NW, KS, CH)
_SELF_DST3 = _SELF_IDS.reshape(NW, KS, CH)


def _seg_common(y_hbm, src_hbm, dst_hbm, acc_s, y_s, src_v, dst_v, rows_v,
                zbuf_v, sem_g, s, wid, y_rows_per_sub):
    pltpu.sync_copy(src_hbm.at[wid], src_v)
    pltpu.sync_copy(dst_hbm.at[wid], dst_v)
    pltpu.sync_copy(y_hbm.at[pl.ds(s * y_rows_per_sub, y_rows_per_sub)],
                    y_s.at[pl.ds(s * y_rows_per_sub, y_rows_per_sub)])

    zeros16 = jnp.zeros((16,), jnp.float32)

    def _zfill(i, carry):
        zbuf_v[i // 2, pl.ds((i % 2) * 16, 16)] = zeros16
        return carry

    lax.fori_loop(0, ZROWS * 2, _zfill, 0)
    for t in range(ACC_PER_SUB // ZROWS):
        pltpu.sync_copy(zbuf_v,
                        acc_s.at[pl.ds(s * ACC_PER_SUB + t * ZROWS, ZROWS)])
    plsc.subcore_barrier()

    pltpu.async_copy(y_s.at[src_v.at[0]], rows_v.at[0], sem_g)

    def _pair(g, carry):
        for b in range(2):
            j = 2 * g + b
            pltpu.make_async_copy(y_s.at[src_v.at[j]], rows_v.at[b],
                                  sem_g).wait()
            pltpu.async_copy(y_s.at[src_v.at[j + 1]], rows_v.at[1 - b], sem_g)
            pltpu.sync_copy(rows_v.at[b], acc_s.at[dst_v.at[j]], add=True)
        return carry

    lax.fori_loop(0, K // 2, _pair, 0)
    pltpu.make_async_copy(y_s.at[src_v.at[K - 1]], rows_v.at[0], sem_g).wait()
    pltpu.async_copy(y_s.at[src_v.at[K]], rows_v.at[1], sem_g)
    pltpu.sync_copy(rows_v.at[0], acc_s.at[dst_v.at[K - 1]], add=True)
    pltpu.make_async_copy(y_s.at[src_v.at[K]], rows_v.at[1], sem_g).wait()


def _seg_writeback(p_hbm, acc_s, c, s):
    plsc.subcore_barrier()
    pltpu.sync_copy(acc_s.at[pl.ds(s * ACC_PER_SUB, ACC_PER_SUB)],
                    p_hbm.at[c, pl.ds(s * ACC_PER_SUB, ACC_PER_SUB)])


def _seg_body(y_hbm, src_hbm, dst_hbm, p_hbm, acc_s, y_s, src_v, dst_v,
              rows_v, zbuf_v, sem_g):
    c = lax.axis_index("c")
    s = lax.axis_index("s")
    wid = c * NS + s
    _seg_common(y_hbm, src_hbm, dst_hbm, acc_s, y_s, src_v, dst_v, rows_v,
                zbuf_v, sem_g, s, wid, ACC_ROWS // NS)
    _seg_writeback(p_hbm, acc_s, c, s)


def _seg0_body(y_hbm, r_hbm, src_hbm, dst_hbm, ssrc_hbm, sdst_hbm, p_hbm,
               acc_s, y_s, ssrc_v, sdst_v, src_v, dst_v, rows_v,
               zbuf_v, sem_g):
    c = lax.axis_index("c")
    s = lax.axis_index("s")
    wid = c * NS + s
    pltpu.sync_copy(ssrc_hbm.at[wid], ssrc_v)
    pltpu.sync_copy(sdst_hbm.at[wid], sdst_v)
    _seg_common(y_hbm, src_hbm, dst_hbm, acc_s, y_s, src_v, dst_v, rows_v,
                zbuf_v, sem_g, s, wid, YSTG)
    def _schunk(j, carry):
        pltpu.async_copy(r_hbm.at[ssrc_v.at[j]], rows_v.at[0], sem_g).wait()
        pltpu.sync_copy(rows_v.at[0], acc_s.at[sdst_v.at[j]], add=True)
        return carry

    lax.fori_loop(0, KS, _schunk, 0)
    _seg_writeback(p_hbm, acc_s, c, s)


_MESH = dict(core_axis_name="c", subcore_axis_name="s",
             num_cores=NC, num_subcores=NS)


@functools.lru_cache(maxsize=2)
def _seg_kernel(first):
    common_scratch = [
        pltpu.VMEM((KP, CH), jnp.int32),
        pltpu.VMEM((KP, CH), jnp.int32),
        pltpu.VMEM((2, CH, HP), jnp.float32),
        pltpu.VMEM((ZROWS, HP), jnp.float32),
        pltpu.SemaphoreType.DMA,
    ]
    if first:
        return pl.kernel(
            _seg0_body,
            out_type=jax.ShapeDtypeStruct((NC, ACC_ROWS, HP), jnp.float32),
            mesh=plsc.VectorSubcoreMesh(**_MESH),
            scratch_types=[
                pltpu.VMEM_SHARED((ACC_ROWS, HP), jnp.float32),
                pltpu.VMEM_SHARED((NN, HP), jnp.float32),
                pltpu.VMEM((KS, CH), jnp.int32),
                pltpu.VMEM((KS, CH), jnp.int32),
            ] + common_scratch,
            compiler_params=pltpu.CompilerParams(use_tc_tiling_on_sc=False),
        )
    return pl.kernel(
        _seg_body,
        out_type=jax.ShapeDtypeStruct((NC, ACC_ROWS, HP), jnp.float32),
        mesh=plsc.VectorSubcoreMesh(**_MESH),
        scratch_types=[
            pltpu.VMEM_SHARED((ACC_ROWS, HP), jnp.float32),
            pltpu.VMEM_SHARED((ACC_ROWS, HP), jnp.float32),
        ] + common_scratch,
        compiler_params=pltpu.CompilerParams(use_tc_tiling_on_sc=False),
    )


_BLK_A = 2000


def _stage_a_body(x_ref, wr_ref, wq_ref, t_ref, r_ref):
    x = x_ref[...]
    t_ref[...] = jnp.dot(x, wr_ref[...], preferred_element_type=jnp.float32)
    r_ref[...] = jnp.dot(x, wq_ref[...], preferred_element_type=jnp.float32)


def _stage_a(x, wr, wq):
    grid = (NN // _BLK_A,)
    return pl.pallas_call(
        _stage_a_body,
        grid=grid,
        in_specs=[
            pl.BlockSpec((_BLK_A, D_IN), lambda i: (i, 0)),
            pl.BlockSpec((D_IN, HP), lambda i: (0, 0)),
            pl.BlockSpec((D_IN, HP), lambda i: (0, 0)),
        ],
        out_specs=[pl.BlockSpec((_BLK_A, HP), lambda i: (i, 0))] * 2,
        out_shape=[jax.ShapeDtypeStruct((NN, HP), jnp.float32)] * 2,
    )(x, wr, wq)


def _stage_b1_body(p_ref, b_ref, h_ref):
    a = p_ref[0] + p_ref[1]
    h_ref[...] = jnp.maximum(a + b_ref[...], 0.0)


def _stage_b1(p, b4):
    grid = (G // GB,)
    return pl.pallas_call(
        _stage_b1_body,
        grid=grid,
        in_specs=[
            pl.BlockSpec((NC, GB, 8, 128), lambda i: (0, i, 0, 0)),
            pl.BlockSpec((1, 128), lambda i: (0, 0)),
        ],
        out_specs=pl.BlockSpec((GB, 8, 128), lambda i: (i, 0, 0)),
        out_shape=jax.ShapeDtypeStruct((G, 8, 128), jnp.float32),
    )(p, b4)


def _stage_b_body(p_ref, h_ref, wr_ref, wq_ref, b_ref, o_ref):
    a = (p_ref[0] + p_ref[1]).reshape(GB * 8, 128)
    hp = h_ref[...].reshape(GB * 8, 128)
    o = jnp.maximum(
        jnp.dot(a, wr_ref[...], preferred_element_type=jnp.float32)
        + b_ref[...]
        + jnp.dot(hp, wq_ref[...], preferred_element_type=jnp.float32),
        0.0,
    )
    o_ref[...] = o.reshape(GB, 8, 128)


def _stage_b(p, h, wr4, wq4, b4):
    grid = (G // GB,)
    return pl.pallas_call(
        _stage_b_body,
        grid=grid,
        in_specs=[
            pl.BlockSpec((NC, GB, 8, 128), lambda i: (0, i, 0, 0)),
            pl.BlockSpec((GB, 8, 128), lambda i: (i, 0, 0)),
            pl.BlockSpec((128, 128), lambda i: (0, 0)),
            pl.BlockSpec((128, 128), lambda i: (0, 0)),
            pl.BlockSpec((1, 128), lambda i: (0, 0)),
        ],
        out_specs=pl.BlockSpec((GB, 8, 128), lambda i: (i, 0, 0)),
        out_shape=jax.ShapeDtypeStruct((G, 8, 128), jnp.float32),
    )(p, h, wr4, wq4, b4)


_CBLK = 125
_NBLK_C = GG // _CBLK


def _stage_c_body(p_ref, h3_ref, wr_ref, bo_ref, wq_ref, wl1_ref, bl1_ref,
                  wl2_ref, bl2_ref, wh1_ref, bh1_ref, wh2_ref, bh2_ref,
                  wh3_ref, bh3_ref, out_ref, acc):
    i = pl.program_id(0)
    a = (p_ref[0] + p_ref[1]).reshape(_CBLK * 8, 128)
    hp = h3_ref[...].reshape(_CBLK * 8, 128)
    h4 = jnp.maximum(
        jnp.dot(a, wr_ref[...], preferred_element_type=jnp.float32)
        + bo_ref[...]
        + jnp.dot(hp, wq_ref[...], preferred_element_type=jnp.float32),
        0.0,
    )

    rows = _CBLK * 8
    row_iota = lax.broadcasted_iota(jnp.int32, (rows, 1), 0)
    s0 = jnp.zeros((1, D_OUT), jnp.float32)
    s1 = jnp.zeros((1, D_OUT), jnp.float32)
    for q in range(4):
        nid = (i * rows + row_iota) * 4 + q
        hq = h4[:, q * D_OUT:(q + 1) * D_OUT]
        m0 = nid < N
        s0 = s0 + jnp.sum(jnp.where(m0, hq, 0.0), axis=0, keepdims=True)
        s1 = s1 + jnp.sum(jnp.where(m0, 0.0, hq), axis=0, keepdims=True)

    @pl.when(i == 0)
    def _():
        acc[0:1, :] = s0
        acc[1:2, :] = s1

    @pl.when(i > 0)
    def _():
        acc[0:1, :] = acc[0:1, :] + s0
        acc[1:2, :] = acc[1:2, :] + s1

    @pl.when(i == _NBLK_C - 1)
    def _():
        m = jnp.maximum(
            jnp.dot(acc[...], wl1_ref[...], preferred_element_type=jnp.float32)
            + bl1_ref[...], 0.0)
        m = jnp.maximum(
            jnp.dot(m, wl2_ref[...], preferred_element_type=jnp.float32)
            + bl2_ref[...], 0.0)
        z = (jnp.dot(m[0:1, :], wh1_ref[0:D_OUT, :],
                     preferred_element_type=jnp.float32)
             + jnp.dot(m[1:2, :], wh1_ref[D_OUT:2 * D_OUT, :],
                       preferred_element_type=jnp.float32)
             + bh1_ref[...])
        z = jnp.maximum(z, 0.0)
        z = jnp.maximum(
            jnp.dot(z, wh2_ref[...], preferred_element_type=jnp.float32)
            + bh2_ref[...], 0.0)
        z = (jnp.dot(z, wh3_ref[...], preferred_element_type=jnp.float32)
             + bh3_ref[...])
        out_ref[...] = 1.0 / (1.0 + jnp.exp(-z))


def _stage_c(p, h3, wr4, bo4, wq4, wl1, bl1, wl2, bl2, wh1, bh1, wh2, bh2,
             wh3, bh3):
    grid = (_NBLK_C,)

    def _full(shape):
        nd = len(shape)
        return pl.BlockSpec(shape, lambda i, _nd=nd: (0,) * _nd)

    return pl.pallas_call(
        _stage_c_body,
        grid=grid,
        in_specs=[
            pl.BlockSpec((NC, _CBLK, 8, 128), lambda i: (0, i, 0, 0)),
            pl.BlockSpec((_CBLK, 8, 128), lambda i: (i, 0, 0)),
            _full((128, 4 * D_OUT)),
            _full((1, 4 * D_OUT)),
            _full((128, 4 * D_OUT)),
            _full((D_OUT, D_OUT)),
            _full((1, D_OUT)),
            _full((D_OUT, D_OUT)),
            _full((1, D_OUT)),
            _full((2 * D_OUT, 10)),
            _full((1, 10)),
            _full((10, 10)),
            _full((1, 10)),
            _full((10, 1)),
            _full((1, 1)),
        ],
        out_specs=pl.BlockSpec((1, 1), lambda i: (0, 0)),
        out_shape=jax.ShapeDtypeStruct((1, 1), jnp.float32),
        scratch_shapes=[pltpu.VMEM((2, D_OUT), jnp.float32)],
    )(p, h3, wr4, bo4, wq4, wl1, bl1, wl2, bl2, wh1, bh1, wh2, bh2, wh3, bh3)


def _pad_cols(w, width=HP):
    return jnp.pad(w, ((0, 0), (0, width - w.shape[1])))


def _pad_rows(w, height=HP):
    return jnp.pad(w, ((0, height - w.shape[0]), (0, 0)))


def _bd4(w):
    return jnp.kron(jnp.eye(4, dtype=w.dtype), w)


def kernel(mol_1_graph, mol_1_nodes, mol_2_graph, mol_2_nodes, params):
    pr = params
    wr_in = _pad_cols(pr['conv_in']['W_rel'])
    wq_in = _pad_cols(pr['conv_in']['W_root'])
    b_in4 = jnp.tile(_pad_cols(pr['conv_in']['b'][None]), (1, 4))
    li1, li2 = pr['conv_internal']
    wr1 = _bd4(_pad_cols(_pad_rows(li1['W_rel'])))
    wq1 = _bd4(_pad_cols(_pad_rows(li1['W_root'])))
    b14 = jnp.tile(_pad_cols(li1['b'][None]), (1, 4))
    wr2 = _bd4(_pad_cols(_pad_rows(li2['W_rel'])))
    wq2 = _bd4(_pad_cols(_pad_rows(li2['W_root'])))
    b24 = jnp.tile(_pad_cols(li2['b'][None]), (1, 4))
    wr_out4 = _bd4(_pad_rows(pr['conv_out']['W_rel']))
    wq_out4 = _bd4(_pad_rows(pr['conv_out']['W_root']))
    b_out4 = jnp.tile(pr['conv_out']['b'][None], (1, 4))
    lo1, lo2 = pr['linear_output']
    wh1 = pr['linear_1']['W']
    bh1 = pr['linear_1']['b'][None]
    wh2 = pr['linear_2']['W']
    bh2 = pr['linear_2']['b'][None]
    wh3 = pr['linear_3']['W']
    bh3 = pr['linear_3']['b'][None]

    x = jnp.concatenate([mol_1_nodes, mol_2_nodes], axis=0)
    src = jnp.concatenate([
        mol_1_graph[0], mol_2_graph[0] + N,
        jnp.zeros((EPAD,), jnp.int32),
    ])
    dst = jnp.concatenate([
        mol_1_graph[1], mol_2_graph[1] + N,
        jnp.full((EPAD,), NN, jnp.int32),
    ])
    src3 = jnp.concatenate([
        src.reshape(NW, K, CH),
        jnp.zeros((NW, KP - K, CH), jnp.int32),
    ], axis=1)
    dst3 = jnp.concatenate([
        dst.reshape(NW, K, CH),
        jnp.full((NW, KP - K, CH), NN, jnp.int32),
    ], axis=1)
    ssrc3 = jnp.asarray(_SELF_SRC3)
    sdst3 = jnp.asarray(_SELF_DST3)

    t0, r0 = _stage_a(x, wr_in, wq_in)
    p0 = _seg_kernel(True)(t0, r0, src3, dst3, ssrc3, sdst3)
    p0 = p0.reshape(NC, G, 8, 128)
    h1 = _stage_b1(p0, b_in4)
    p1 = _seg_kernel(False)(h1.reshape(ACC_ROWS, HP), src3, dst3)
    h2 = _stage_b(p1.reshape(NC, G, 8, 128), h1, wr1, wq1, b14)
    p2 = _seg_kernel(False)(h2.reshape(ACC_ROWS, HP), src3, dst3)
    h3 = _stage_b(p2.reshape(NC, G, 8, 128), h2, wr2, wq2, b24)
    p3 = _seg_kernel(False)(h3.reshape(ACC_ROWS, HP), src3, dst3)
    out = _stage_c(p3.reshape(NC, G, 8, 128), h3, wr_out4, b_out4, wq_out4,
                   lo1['W'], lo1['b'][None], lo2['W'], lo2['b'][None],
                   wh1, bh1, wh2, bh2, wh3, bh3)
    return out.reshape((1,))


# speedup vs baseline: 6.6005x; 1.0411x over previous
import functools

import jax
import jax.numpy as jnp
from jax import lax
from jax.experimental import pallas as pl
from jax.experimental.pallas import tpu as pltpu
from jax.experimental.pallas import tpu_sc as plsc

N = 10000
E = 160000
D_IN = 256
HID = 20
HP = 32
D_OUT = 128
NN = 2 * N

NC, NS = 2, 16
NW = NC * NS
CH = 128
E2 = 2 * E
K = -(-E2 // (NW * CH))
KP = K + 1
EP = NW * K * CH
EPAD = EP - E2

ACC_ROWS = 20480
ZROWS = 160
ACC_PER_SUB = ACC_ROWS // NS

G = ACC_ROWS * HP // 1024
GB = 64
GG = 625


def _seg_common(y_hbm, src_hbm, dst_hbm, acc_s, y_s, src_v, dst_v, rows_v,
                zbuf_v, sem_g, s, wid, y_rows_per_sub):
    pltpu.sync_copy(src_hbm.at[wid], src_v)
    pltpu.sync_copy(dst_hbm.at[wid], dst_v)
    pltpu.sync_copy(y_hbm.at[pl.ds(s * y_rows_per_sub, y_rows_per_sub)],
                    y_s.at[pl.ds(s * y_rows_per_sub, y_rows_per_sub)])

    zeros16 = jnp.zeros((16,), jnp.float32)

    def _zfill(i, carry):
        zbuf_v[i // 2, pl.ds((i % 2) * 16, 16)] = zeros16
        return carry

    lax.fori_loop(0, ZROWS * 2, _zfill, 0)
    for t in range(ACC_PER_SUB // ZROWS):
        pltpu.sync_copy(zbuf_v,
                        acc_s.at[pl.ds(s * ACC_PER_SUB + t * ZROWS, ZROWS)])
    plsc.subcore_barrier()

    pltpu.async_copy(y_s.at[src_v.at[0]], rows_v.at[0], sem_g)

    def _pair(g, carry):
        for b in range(2):
            j = 2 * g + b
            pltpu.make_async_copy(y_s.at[src_v.at[j]], rows_v.at[b],
                                  sem_g).wait()
            pltpu.async_copy(y_s.at[src_v.at[j + 1]], rows_v.at[1 - b], sem_g)
            pltpu.sync_copy(rows_v.at[b], acc_s.at[dst_v.at[j]], add=True)
        return carry

    lax.fori_loop(0, K // 2, _pair, 0)
    pltpu.make_async_copy(y_s.at[src_v.at[K - 1]], rows_v.at[0], sem_g).wait()
    pltpu.async_copy(y_s.at[src_v.at[K]], rows_v.at[1], sem_g)
    pltpu.sync_copy(rows_v.at[0], acc_s.at[dst_v.at[K - 1]], add=True)
    pltpu.make_async_copy(y_s.at[src_v.at[K]], rows_v.at[1], sem_g).wait()


def _seg_writeback(p_hbm, acc_s, c, s):
    plsc.subcore_barrier()
    pltpu.sync_copy(acc_s.at[pl.ds(s * ACC_PER_SUB, ACC_PER_SUB)],
                    p_hbm.at[c, pl.ds(s * ACC_PER_SUB, ACC_PER_SUB)])


def _make_seg_body(y_rows):
    rows_per_sub = y_rows // NS

    def _seg_body(y_hbm, src_hbm, dst_hbm, p_hbm, acc_s, y_s, src_v, dst_v,
                  rows_v, zbuf_v, sem_g):
        c = lax.axis_index("c")
        s = lax.axis_index("s")
        wid = c * NS + s
        _seg_common(y_hbm, src_hbm, dst_hbm, acc_s, y_s, src_v, dst_v, rows_v,
                    zbuf_v, sem_g, s, wid, rows_per_sub)
        _seg_writeback(p_hbm, acc_s, c, s)

    return _seg_body


_MESH = dict(core_axis_name="c", subcore_axis_name="s",
             num_cores=NC, num_subcores=NS)


@functools.lru_cache(maxsize=2)
def _seg_kernel(y_rows):
    return pl.kernel(
        _make_seg_body(y_rows),
        out_type=jax.ShapeDtypeStruct((NC, ACC_ROWS, HP), jnp.float32),
        mesh=plsc.VectorSubcoreMesh(**_MESH),
        scratch_types=[
            pltpu.VMEM_SHARED((ACC_ROWS, HP), jnp.float32),
            pltpu.VMEM_SHARED((y_rows, HP), jnp.float32),
            pltpu.VMEM((KP, CH), jnp.int32),
            pltpu.VMEM((KP, CH), jnp.int32),
            pltpu.VMEM((2, CH, HP), jnp.float32),
            pltpu.VMEM((ZROWS, HP), jnp.float32),
            pltpu.SemaphoreType.DMA,
        ],
        compiler_params=pltpu.CompilerParams(use_tc_tiling_on_sc=False),
    )


_XROWS = NN // 4
_ABLK = 1000


def _stage_a_body(x_ref, w_ref, o_ref):
    o = jnp.dot(x_ref[...], w_ref[...], preferred_element_type=jnp.float32)
    o_ref[...] = o.reshape(_ABLK // 8, 8, 128)


def _stage_a(xp, w4):
    grid = (_XROWS // _ABLK,)
    return pl.pallas_call(
        _stage_a_body,
        grid=grid,
        in_specs=[
            pl.BlockSpec((_ABLK, 4 * D_IN), lambda i: (i, 0)),
            pl.BlockSpec((4 * D_IN, 128), lambda i: (0, 0)),
        ],
        out_specs=pl.BlockSpec((_ABLK // 8, 8, 128), lambda i: (i, 0, 0)),
        out_shape=jax.ShapeDtypeStruct((G, 8, 128), jnp.float32),
    )(xp, w4)


def _stage_b1_body(p_ref, r_ref, b_ref, h_ref):
    a = p_ref[0] + p_ref[1]
    h_ref[...] = jnp.maximum(a + r_ref[...] + b_ref[...], 0.0)


def _stage_b1(p, r, b4):
    grid = (G // GB,)
    return pl.pallas_call(
        _stage_b1_body,
        grid=grid,
        in_specs=[
            pl.BlockSpec((NC, GB, 8, 128), lambda i: (0, i, 0, 0)),
            pl.BlockSpec((GB, 8, 128), lambda i: (i, 0, 0)),
            pl.BlockSpec((1, 128), lambda i: (0, 0)),
        ],
        out_specs=pl.BlockSpec((GB, 8, 128), lambda i: (i, 0, 0)),
        out_shape=jax.ShapeDtypeStruct((G, 8, 128), jnp.float32),
    )(p, r, b4)


def _stage_b_body(p_ref, h_ref, wr_ref, wq_ref, b_ref, o_ref):
    a = (p_ref[0] + p_ref[1]).reshape(GB * 8, 128)
    hp = h_ref[...].reshape(GB * 8, 128)
    o = jnp.maximum(
        jnp.dot(a, wr_ref[...], preferred_element_type=jnp.float32)
        + b_ref[...]
        + jnp.dot(hp, wq_ref[...], preferred_element_type=jnp.float32),
        0.0,
    )
    o_ref[...] = o.reshape(GB, 8, 128)


def _stage_b(p, h, wr4, wq4, b4):
    grid = (G // GB,)
    return pl.pallas_call(
        _stage_b_body,
        grid=grid,
        in_specs=[
            pl.BlockSpec((NC, GB, 8, 128), lambda i: (0, i, 0, 0)),
            pl.BlockSpec((GB, 8, 128), lambda i: (i, 0, 0)),
            pl.BlockSpec((128, 128), lambda i: (0, 0)),
            pl.BlockSpec((128, 128), lambda i: (0, 0)),
            pl.BlockSpec((1, 128), lambda i: (0, 0)),
        ],
        out_specs=pl.BlockSpec((GB, 8, 128), lambda i: (i, 0, 0)),
        out_shape=jax.ShapeDtypeStruct((G, 8, 128), jnp.float32),
    )(p, h, wr4, wq4, b4)


_CBLK = 125
_NBLK_C = GG // _CBLK


def _stage_c_body(p_ref, h3_ref, wr_ref, bo_ref, wq_ref, wl1_ref, bl1_ref,
                  wl2_ref, bl2_ref, wh1_ref, bh1_ref, wh2_ref, bh2_ref,
                  wh3_ref, bh3_ref, out_ref, acc):
    i = pl.program_id(0)
    a = (p_ref[0] + p_ref[1]).reshape(_CBLK * 8, 128)
    hp = h3_ref[...].reshape(_CBLK * 8, 128)
    h4 = jnp.maximum(
        jnp.dot(a, wr_ref[...], preferred_element_type=jnp.float32)
        + bo_ref[...]
        + jnp.dot(hp, wq_ref[...], preferred_element_type=jnp.float32),
        0.0,
    )

    rows = _CBLK * 8
    row_iota = lax.broadcasted_iota(jnp.int32, (rows, 1), 0)
    s0 = jnp.zeros((1, D_OUT), jnp.float32)
    s1 = jnp.zeros((1, D_OUT), jnp.float32)
    for q in range(4):
        nid = (i * rows + row_iota) * 4 + q
        hq = h4[:, q * D_OUT:(q + 1) * D_OUT]
        m0 = nid < N
        s0 = s0 + jnp.sum(jnp.where(m0, hq, 0.0), axis=0, keepdims=True)
        s1 = s1 + jnp.sum(jnp.where(m0, 0.0, hq), axis=0, keepdims=True)

    @pl.when(i == 0)
    def _():
        acc[0:1, :] = s0
        acc[1:2, :] = s1

    @pl.when(i > 0)
    def _():
        acc[0:1, :] = acc[0:1, :] + s0
        acc[1:2, :] = acc[1:2, :] + s1

    @pl.when(i == _NBLK_C - 1)
    def _():
        m = jnp.maximum(
            jnp.dot(acc[...], wl1_ref[...], preferred_element_type=jnp.float32)
            + bl1_ref[...], 0.0)
        m = jnp.maximum(
            jnp.dot(m, wl2_ref[...], preferred_element_type=jnp.float32)
            + bl2_ref[...], 0.0)
        z = (jnp.dot(m[0:1, :], wh1_ref[0:D_OUT, :],
                     preferred_element_type=jnp.float32)
             + jnp.dot(m[1:2, :], wh1_ref[D_OUT:2 * D_OUT, :],
                       preferred_element_type=jnp.float32)
             + bh1_ref[...])
        z = jnp.maximum(z, 0.0)
        z = jnp.maximum(
            jnp.dot(z, wh2_ref[...], preferred_element_type=jnp.float32)
            + bh2_ref[...], 0.0)
        z = (jnp.dot(z, wh3_ref[...], preferred_element_type=jnp.float32)
             + bh3_ref[...])
        out_ref[...] = 1.0 / (1.0 + jnp.exp(-z))


def _stage_c(p, h3, wr4, bo4, wq4, wl1, bl1, wl2, bl2, wh1, bh1, wh2, bh2,
             wh3, bh3):
    grid = (_NBLK_C,)

    def _full(shape):
        nd = len(shape)
        return pl.BlockSpec(shape, lambda i, _nd=nd: (0,) * _nd)

    return pl.pallas_call(
        _stage_c_body,
        grid=grid,
        in_specs=[
            pl.BlockSpec((NC, _CBLK, 8, 128), lambda i: (0, i, 0, 0)),
            pl.BlockSpec((_CBLK, 8, 128), lambda i: (i, 0, 0)),
            _full((128, 4 * D_OUT)),
            _full((1, 4 * D_OUT)),
            _full((128, 4 * D_OUT)),
            _full((D_OUT, D_OUT)),
            _full((1, D_OUT)),
            _full((D_OUT, D_OUT)),
            _full((1, D_OUT)),
            _full((2 * D_OUT, 10)),
            _full((1, 10)),
            _full((10, 10)),
            _full((1, 10)),
            _full((10, 1)),
            _full((1, 1)),
        ],
        out_specs=pl.BlockSpec((1, 1), lambda i: (0, 0)),
        out_shape=jax.ShapeDtypeStruct((1, 1), jnp.float32),
        scratch_shapes=[pltpu.VMEM((2, D_OUT), jnp.float32)],
    )(p, h3, wr4, bo4, wq4, wl1, bl1, wl2, bl2, wh1, bh1, wh2, bh2, wh3, bh3)


def _pad_cols(w, width=HP):
    return jnp.pad(w, ((0, 0), (0, width - w.shape[1])))


def _pad_rows(w, height=HP):
    return jnp.pad(w, ((0, height - w.shape[0]), (0, 0)))


def _bd4(w):
    return jnp.kron(jnp.eye(4, dtype=w.dtype), w)


def kernel(mol_1_graph, mol_1_nodes, mol_2_graph, mol_2_nodes, params):
    pr = params
    wr_in4 = _bd4(_pad_cols(pr['conv_in']['W_rel']))
    wq_in4 = _bd4(_pad_cols(pr['conv_in']['W_root']))
    b_in4 = jnp.tile(_pad_cols(pr['conv_in']['b'][None]), (1, 4))
    li1, li2 = pr['conv_internal']
    wr1 = _bd4(_pad_cols(_pad_rows(li1['W_rel'])))
    wq1 = _bd4(_pad_cols(_pad_rows(li1['W_root'])))
    b14 = jnp.tile(_pad_cols(li1['b'][None]), (1, 4))
    wr2 = _bd4(_pad_cols(_pad_rows(li2['W_rel'])))
    wq2 = _bd4(_pad_cols(_pad_rows(li2['W_root'])))
    b24 = jnp.tile(_pad_cols(li2['b'][None]), (1, 4))
    wr_out4 = _bd4(_pad_rows(pr['conv_out']['W_rel']))
    wq_out4 = _bd4(_pad_rows(pr['conv_out']['W_root']))
    b_out4 = jnp.tile(pr['conv_out']['b'][None], (1, 4))
    lo1, lo2 = pr['linear_output']
    wh1 = pr['linear_1']['W']
    bh1 = pr['linear_1']['b'][None]
    wh2 = pr['linear_2']['W']
    bh2 = pr['linear_2']['b'][None]
    wh3 = pr['linear_3']['W']
    bh3 = pr['linear_3']['b'][None]

    xp = jnp.concatenate([mol_1_nodes, mol_2_nodes],
                         axis=0).reshape(_XROWS, 4 * D_IN)
    src = jnp.concatenate([
        mol_1_graph[0], mol_2_graph[0] + N,
        jnp.zeros((EPAD,), jnp.int32),
    ])
    dst = jnp.concatenate([
        mol_1_graph[1], mol_2_graph[1] + N,
        jnp.full((EPAD,), NN, jnp.int32),
    ])
    src3 = jnp.concatenate([
        src.reshape(NW, K, CH),
        jnp.zeros((NW, KP - K, CH), jnp.int32),
    ], axis=1)
    dst3 = jnp.concatenate([
        dst.reshape(NW, K, CH),
        jnp.full((NW, KP - K, CH), NN, jnp.int32),
    ], axis=1)
    t0 = _stage_a(xp, wr_in4)
    p0 = _seg_kernel(ACC_ROWS)(t0.reshape(ACC_ROWS, HP), src3, dst3)
    r0 = _stage_a(xp, wq_in4)
    h1 = _stage_b1(p0.reshape(NC, G, 8, 128), r0, b_in4)
    p1 = _seg_kernel(ACC_ROWS)(h1.reshape(ACC_ROWS, HP), src3, dst3)
    h2 = _stage_b(p1.reshape(NC, G, 8, 128), h1, wr1, wq1, b14)
    p2 = _seg_kernel(ACC_ROWS)(h2.reshape(ACC_ROWS, HP), src3, dst3)
    h3 = _stage_b(p2.reshape(NC, G, 8, 128), h2, wr2, wq2, b24)
    p3 = _seg_kernel(ACC_ROWS)(h3.reshape(ACC_ROWS, HP), src3, dst3)
    out = _stage_c(p3.reshape(NC, G, 8, 128), h3, wr_out4, b_out4, wq_out4,
                   lo1['W'], lo1['b'][None], lo2['W'], lo2['b'][None],
                   wh1, bh1, wh2, bh2, wh3, bh3)
    return out.reshape((1,))


# speedup vs baseline: 6.7761x; 1.0266x over previous
import functools

import jax
import jax.numpy as jnp
from jax import lax
from jax.experimental import pallas as pl
from jax.experimental.pallas import tpu as pltpu
from jax.experimental.pallas import tpu_sc as plsc

N = 10000
E = 160000
D_IN = 256
HID = 20
HP = 32
D_OUT = 128
NN = 2 * N

NC, NS = 2, 16
NW = NC * NS
CH = 128
E2 = 2 * E
K = -(-E2 // (NW * CH))
KP = K + 1
EP = NW * K * CH
EPAD = EP - E2

ACC_ROWS = 20480
ZROWS = 160
ACC_PER_SUB = ACC_ROWS // NS

G = ACC_ROWS * HP // 1024
GB = 64
GG = 625


def _seg_common(y_hbm, src_hbm, dst_hbm, acc_s, y_s, src_v, dst_v, rows_v,
                zbuf_v, sem_g, sem_s, s, wid, y_rows_per_sub):
    pltpu.sync_copy(src_hbm.at[wid], src_v)
    pltpu.sync_copy(dst_hbm.at[wid], dst_v)
    pltpu.sync_copy(y_hbm.at[pl.ds(s * y_rows_per_sub, y_rows_per_sub)],
                    y_s.at[pl.ds(s * y_rows_per_sub, y_rows_per_sub)])

    zeros16 = jnp.zeros((16,), jnp.float32)

    def _zfill(i, carry):
        zbuf_v[i // 2, pl.ds((i % 2) * 16, 16)] = zeros16
        return carry

    lax.fori_loop(0, ZROWS * 2, _zfill, 0)
    for t in range(ACC_PER_SUB // ZROWS):
        pltpu.sync_copy(zbuf_v,
                        acc_s.at[pl.ds(s * ACC_PER_SUB + t * ZROWS, ZROWS)])
    plsc.subcore_barrier()

    sems = (sem_g, sem_s)
    pltpu.async_copy(y_s.at[src_v.at[0]], rows_v.at[0], sems[0])
    pltpu.async_copy(y_s.at[src_v.at[1]], rows_v.at[1], sems[1])

    def _six(g, carry):
        for t in range(6):
            j = 6 * g + t
            b = t % 3
            sm = sems[t % 2]
            smn = sems[t % 2]
            pltpu.make_async_copy(y_s.at[src_v.at[j]], rows_v.at[b],
                                  sm).wait()
            pltpu.async_copy(y_s.at[src_v.at[j + 2]], rows_v.at[(b + 2) % 3],
                             smn)
            pltpu.sync_copy(rows_v.at[b], acc_s.at[dst_v.at[j]], add=True)
        return carry

    lax.fori_loop(0, K // 6, _six, 0)
    pltpu.make_async_copy(y_s.at[src_v.at[K - 1]], rows_v.at[0],
                          sems[0]).wait()
    pltpu.sync_copy(rows_v.at[0], acc_s.at[dst_v.at[K - 1]], add=True)
    pltpu.make_async_copy(y_s.at[src_v.at[K]], rows_v.at[1], sems[1]).wait()


def _seg_writeback(p_hbm, acc_s, c, s):
    plsc.subcore_barrier()
    pltpu.sync_copy(acc_s.at[pl.ds(s * ACC_PER_SUB, ACC_PER_SUB)],
                    p_hbm.at[c, pl.ds(s * ACC_PER_SUB, ACC_PER_SUB)])


def _make_seg_body(y_rows):
    rows_per_sub = y_rows // NS

    def _seg_body(y_hbm, src_hbm, dst_hbm, p_hbm, acc_s, y_s, src_v, dst_v,
                  rows_v, zbuf_v, sem_g, sem_s):
        c = lax.axis_index("c")
        s = lax.axis_index("s")
        wid = c * NS + s
        _seg_common(y_hbm, src_hbm, dst_hbm, acc_s, y_s, src_v, dst_v, rows_v,
                    zbuf_v, sem_g, sem_s, s, wid, rows_per_sub)
        _seg_writeback(p_hbm, acc_s, c, s)

    return _seg_body


_MESH = dict(core_axis_name="c", subcore_axis_name="s",
             num_cores=NC, num_subcores=NS)


@functools.lru_cache(maxsize=2)
def _seg_kernel(y_rows):
    return pl.kernel(
        _make_seg_body(y_rows),
        out_type=jax.ShapeDtypeStruct((NC, ACC_ROWS, HP), jnp.float32),
        mesh=plsc.VectorSubcoreMesh(**_MESH),
        scratch_types=[
            pltpu.VMEM_SHARED((ACC_ROWS, HP), jnp.float32),
            pltpu.VMEM_SHARED((y_rows, HP), jnp.float32),
            pltpu.VMEM((KP, CH), jnp.int32),
            pltpu.VMEM((KP, CH), jnp.int32),
            pltpu.VMEM((3, CH, HP), jnp.float32),
            pltpu.VMEM((ZROWS, HP), jnp.float32),
            pltpu.SemaphoreType.DMA,
            pltpu.SemaphoreType.DMA,
        ],
        compiler_params=pltpu.CompilerParams(use_tc_tiling_on_sc=False),
    )


_XROWS = NN // 4
_ABLK = 1000


def _stage_a_body(x_ref, w_ref, o_ref):
    o = jnp.dot(x_ref[...], w_ref[...], preferred_element_type=jnp.float32)
    o_ref[...] = o.reshape(_ABLK // 8, 8, 128)


def _stage_a(xp, w4):
    grid = (_XROWS // _ABLK,)
    return pl.pallas_call(
        _stage_a_body,
        grid=grid,
        in_specs=[
            pl.BlockSpec((_ABLK, 4 * D_IN), lambda i: (i, 0)),
            pl.BlockSpec((4 * D_IN, 128), lambda i: (0, 0)),
        ],
        out_specs=pl.BlockSpec((_ABLK // 8, 8, 128), lambda i: (i, 0, 0)),
        out_shape=jax.ShapeDtypeStruct((G, 8, 128), jnp.float32),
    )(xp, w4)


def _stage_b1_body(p_ref, r_ref, b_ref, h_ref):
    a = p_ref[0] + p_ref[1]
    h_ref[...] = jnp.maximum(a + r_ref[...] + b_ref[...], 0.0)


def _stage_b1(p, r, b4):
    grid = (G // GB,)
    return pl.pallas_call(
        _stage_b1_body,
        grid=grid,
        in_specs=[
            pl.BlockSpec((NC, GB, 8, 128), lambda i: (0, i, 0, 0)),
            pl.BlockSpec((GB, 8, 128), lambda i: (i, 0, 0)),
            pl.BlockSpec((1, 128), lambda i: (0, 0)),
        ],
        out_specs=pl.BlockSpec((GB, 8, 128), lambda i: (i, 0, 0)),
        out_shape=jax.ShapeDtypeStruct((G, 8, 128), jnp.float32),
    )(p, r, b4)


def _stage_b_body(p_ref, h_ref, wr_ref, wq_ref, b_ref, o_ref):
    a = (p_ref[0] + p_ref[1]).reshape(GB * 8, 128)
    hp = h_ref[...].reshape(GB * 8, 128)
    o = jnp.maximum(
        jnp.dot(a, wr_ref[...], preferred_element_type=jnp.float32)
        + b_ref[...]
        + jnp.dot(hp, wq_ref[...], preferred_element_type=jnp.float32),
        0.0,
    )
    o_ref[...] = o.reshape(GB, 8, 128)


def _stage_b(p, h, wr4, wq4, b4):
    grid = (G // GB,)
    return pl.pallas_call(
        _stage_b_body,
        grid=grid,
        in_specs=[
            pl.BlockSpec((NC, GB, 8, 128), lambda i: (0, i, 0, 0)),
            pl.BlockSpec((GB, 8, 128), lambda i: (i, 0, 0)),
            pl.BlockSpec((128, 128), lambda i: (0, 0)),
            pl.BlockSpec((128, 128), lambda i: (0, 0)),
            pl.BlockSpec((1, 128), lambda i: (0, 0)),
        ],
        out_specs=pl.BlockSpec((GB, 8, 128), lambda i: (i, 0, 0)),
        out_shape=jax.ShapeDtypeStruct((G, 8, 128), jnp.float32),
    )(p, h, wr4, wq4, b4)


_CBLK = 125
_NBLK_C = GG // _CBLK


def _stage_c_body(p_ref, h3_ref, wr_ref, bo_ref, wq_ref, wl1_ref, bl1_ref,
                  wl2_ref, bl2_ref, wh1_ref, bh1_ref, wh2_ref, bh2_ref,
                  wh3_ref, bh3_ref, out_ref, acc):
    i = pl.program_id(0)
    a = (p_ref[0] + p_ref[1]).reshape(_CBLK * 8, 128)
    hp = h3_ref[...].reshape(_CBLK * 8, 128)
    h4 = jnp.maximum(
        jnp.dot(a, wr_ref[...], preferred_element_type=jnp.float32)
        + bo_ref[...]
        + jnp.dot(hp, wq_ref[...], preferred_element_type=jnp.float32),
        0.0,
    )

    rows = _CBLK * 8
    row_iota = lax.broadcasted_iota(jnp.int32, (rows, 1), 0)
    s0 = jnp.zeros((1, D_OUT), jnp.float32)
    s1 = jnp.zeros((1, D_OUT), jnp.float32)
    for q in range(4):
        nid = (i * rows + row_iota) * 4 + q
        hq = h4[:, q * D_OUT:(q + 1) * D_OUT]
        m0 = nid < N
        s0 = s0 + jnp.sum(jnp.where(m0, hq, 0.0), axis=0, keepdims=True)
        s1 = s1 + jnp.sum(jnp.where(m0, 0.0, hq), axis=0, keepdims=True)

    @pl.when(i == 0)
    def _():
        acc[0:1, :] = s0
        acc[1:2, :] = s1

    @pl.when(i > 0)
    def _():
        acc[0:1, :] = acc[0:1, :] + s0
        acc[1:2, :] = acc[1:2, :] + s1

    @pl.when(i == _NBLK_C - 1)
    def _():
        m = jnp.maximum(
            jnp.dot(acc[...], wl1_ref[...], preferred_element_type=jnp.float32)
            + bl1_ref[...], 0.0)
        m = jnp.maximum(
            jnp.dot(m, wl2_ref[...], preferred_element_type=jnp.float32)
            + bl2_ref[...], 0.0)
        z = (jnp.dot(m[0:1, :], wh1_ref[0:D_OUT, :],
                     preferred_element_type=jnp.float32)
             + jnp.dot(m[1:2, :], wh1_ref[D_OUT:2 * D_OUT, :],
                       preferred_element_type=jnp.float32)
             + bh1_ref[...])
        z = jnp.maximum(z, 0.0)
        z = jnp.maximum(
            jnp.dot(z, wh2_ref[...], preferred_element_type=jnp.float32)
            + bh2_ref[...], 0.0)
        z = (jnp.dot(z, wh3_ref[...], preferred_element_type=jnp.float32)
             + bh3_ref[...])
        out_ref[...] = 1.0 / (1.0 + jnp.exp(-z))


def _stage_c(p, h3, wr4, bo4, wq4, wl1, bl1, wl2, bl2, wh1, bh1, wh2, bh2,
             wh3, bh3):
    grid = (_NBLK_C,)

    def _full(shape):
        nd = len(shape)
        return pl.BlockSpec(shape, lambda i, _nd=nd: (0,) * _nd)

    return pl.pallas_call(
        _stage_c_body,
        grid=grid,
        in_specs=[
            pl.BlockSpec((NC, _CBLK, 8, 128), lambda i: (0, i, 0, 0)),
            pl.BlockSpec((_CBLK, 8, 128), lambda i: (i, 0, 0)),
            _full((128, 4 * D_OUT)),
            _full((1, 4 * D_OUT)),
            _full((128, 4 * D_OUT)),
            _full((D_OUT, D_OUT)),
            _full((1, D_OUT)),
            _full((D_OUT, D_OUT)),
            _full((1, D_OUT)),
            _full((2 * D_OUT, 10)),
            _full((1, 10)),
            _full((10, 10)),
            _full((1, 10)),
            _full((10, 1)),
            _full((1, 1)),
        ],
        out_specs=pl.BlockSpec((1, 1), lambda i: (0, 0)),
        out_shape=jax.ShapeDtypeStruct((1, 1), jnp.float32),
        scratch_shapes=[pltpu.VMEM((2, D_OUT), jnp.float32)],
    )(p, h3, wr4, bo4, wq4, wl1, bl1, wl2, bl2, wh1, bh1, wh2, bh2, wh3, bh3)


def _pad_cols(w, width=HP):
    return jnp.pad(w, ((0, 0), (0, width - w.shape[1])))


def _pad_rows(w, height=HP):
    return jnp.pad(w, ((0, height - w.shape[0]), (0, 0)))


def _bd4(w):
    return jnp.kron(jnp.eye(4, dtype=w.dtype), w)


def kernel(mol_1_graph, mol_1_nodes, mol_2_graph, mol_2_nodes, params):
    pr = params
    wr_in4 = _bd4(_pad_cols(pr['conv_in']['W_rel']))
    wq_in4 = _bd4(_pad_cols(pr['conv_in']['W_root']))
    b_in4 = jnp.tile(_pad_cols(pr['conv_in']['b'][None]), (1, 4))
    li1, li2 = pr['conv_internal']
    wr1 = _bd4(_pad_cols(_pad_rows(li1['W_rel'])))
    wq1 = _bd4(_pad_cols(_pad_rows(li1['W_root'])))
    b14 = jnp.tile(_pad_cols(li1['b'][None]), (1, 4))
    wr2 = _bd4(_pad_cols(_pad_rows(li2['W_rel'])))
    wq2 = _bd4(_pad_cols(_pad_rows(li2['W_root'])))
    b24 = jnp.tile(_pad_cols(li2['b'][None]), (1, 4))
    wr_out4 = _bd4(_pad_rows(pr['conv_out']['W_rel']))
    wq_out4 = _bd4(_pad_rows(pr['conv_out']['W_root']))
    b_out4 = jnp.tile(pr['conv_out']['b'][None], (1, 4))
    lo1, lo2 = pr['linear_output']
    wh1 = pr['linear_1']['W']
    bh1 = pr['linear_1']['b'][None]
    wh2 = pr['linear_2']['W']
    bh2 = pr['linear_2']['b'][None]
    wh3 = pr['linear_3']['W']
    bh3 = pr['linear_3']['b'][None]

    xp = jnp.concatenate([mol_1_nodes, mol_2_nodes],
                         axis=0).reshape(_XROWS, 4 * D_IN)
    src = jnp.concatenate([
        mol_1_graph[0], mol_2_graph[0] + N,
        jnp.zeros((EPAD,), jnp.int32),
    ])
    dst = jnp.concatenate([
        mol_1_graph[1], mol_2_graph[1] + N,
        jnp.full((EPAD,), NN, jnp.int32),
    ])
    src3 = jnp.concatenate([
        src.reshape(NW, K, CH),
        jnp.zeros((NW, KP - K, CH), jnp.int32),
    ], axis=1)
    dst3 = jnp.concatenate([
        dst.reshape(NW, K, CH),
        jnp.full((NW, KP - K, CH), NN, jnp.int32),
    ], axis=1)
    t0 = _stage_a(xp, wr_in4)
    p0 = _seg_kernel(ACC_ROWS)(t0.reshape(ACC_ROWS, HP), src3, dst3)
    r0 = _stage_a(xp, wq_in4)
    h1 = _stage_b1(p0.reshape(NC, G, 8, 128), r0, b_in4)
    p1 = _seg_kernel(ACC_ROWS)(h1.reshape(ACC_ROWS, HP), src3, dst3)
    h2 = _stage_b(p1.reshape(NC, G, 8, 128), h1, wr1, wq1, b14)
    p2 = _seg_kernel(ACC_ROWS)(h2.reshape(ACC_ROWS, HP), src3, dst3)
    h3 = _stage_b(p2.reshape(NC, G, 8, 128), h2, wr2, wq2, b24)
    p3 = _seg_kernel(ACC_ROWS)(h3.reshape(ACC_ROWS, HP), src3, dst3)
    out = _stage_c(p3.reshape(NC, G, 8, 128), h3, wr_out4, b_out4, wq_out4,
                   lo1['W'], lo1['b'][None], lo2['W'], lo2['b'][None],
                   wh1, bh1, wh2, bh2, wh3, bh3)
    return out.reshape((1,))


# speedup vs baseline: 7.0517x; 1.0407x over previous
import functools

import jax
import jax.numpy as jnp
from jax import lax
from jax.experimental import pallas as pl
from jax.experimental.pallas import tpu as pltpu
from jax.experimental.pallas import tpu_sc as plsc

N = 10000
E = 160000
D_IN = 256
HID = 20
HP = 32
D_OUT = 128
NN = 2 * N

NC, NS = 2, 16
NW = NC * NS
CH = 128
E2 = 2 * E
K = -(-E2 // (NW * CH))
KP = K + 2
EP = NW * K * CH
EPAD = EP - E2

ACC_ROWS = 20480
ZROWS = 160
ACC_PER_SUB = ACC_ROWS // NS

G = ACC_ROWS * HP // 1024
GB = 64
GG = 625


def _seg_common(y_hbm, src_hbm, dst_hbm, acc_s, y_s, src_v, dst_v, rows_v,
                zbuf_v, sem_ga, sem_gb, sem_y, sem_sc, s, wid,
                y_rows_per_sub):
    pltpu.async_copy(src_hbm.at[wid], src_v, sem_ga)
    pltpu.async_copy(dst_hbm.at[wid], dst_v, sem_gb)
    ysl = pl.ds(s * y_rows_per_sub, y_rows_per_sub)
    pltpu.async_copy(y_hbm.at[ysl], y_s.at[ysl], sem_y)

    zeros16 = jnp.zeros((16,), jnp.float32)

    def _zfill(i, carry):
        zbuf_v[i // 2, pl.ds((i % 2) * 16, 16)] = zeros16
        return carry

    lax.fori_loop(0, ZROWS * 2, _zfill, 0)
    pltpu.make_async_copy(src_hbm.at[wid], src_v, sem_ga).wait()
    pltpu.make_async_copy(dst_hbm.at[wid], dst_v, sem_gb).wait()
    pltpu.make_async_copy(y_hbm.at[ysl], y_s.at[ysl], sem_y).wait()

    zsl = [pl.ds(s * ACC_PER_SUB + t * ZROWS, ZROWS)
           for t in range(ACC_PER_SUB // ZROWS)]
    for t in zsl:
        pltpu.async_copy(zbuf_v, acc_s.at[t], sem_y)
    for t in zsl:
        pltpu.make_async_copy(zbuf_v, acc_s.at[t], sem_y).wait()
    plsc.subcore_barrier()

    gsems = (sem_ga, sem_gb)
    pltpu.async_copy(y_s.at[src_v.at[0]], rows_v.at[0], gsems[0])
    pltpu.async_copy(y_s.at[src_v.at[1]], rows_v.at[1], gsems[1])
    pltpu.make_async_copy(y_s.at[src_v.at[0]], rows_v.at[0], gsems[0]).wait()
    pltpu.async_copy(y_s.at[src_v.at[2]], rows_v.at[2], gsems[0])
    pltpu.async_copy(rows_v.at[0], acc_s.at[dst_v.at[0]], sem_sc, add=True)

    def _six(g, carry):
        for t in range(6):
            j = 1 + 6 * g + t
            b = (1 + t) % 3
            sg = gsems[(1 + t) % 2]
            pb = (b + 2) % 3
            pltpu.make_async_copy(y_s.at[src_v.at[j]], rows_v.at[b],
                                  sg).wait()
            pltpu.make_async_copy(rows_v.at[pb], acc_s.at[dst_v.at[j - 1]],
                                  sem_sc).wait()
            pltpu.async_copy(y_s.at[src_v.at[j + 2]], rows_v.at[pb], sg)
            pltpu.async_copy(rows_v.at[b], acc_s.at[dst_v.at[j]], sem_sc,
                             add=True)
        return carry

    lax.fori_loop(0, (K - 1) // 6, _six, 0)
    pltpu.make_async_copy(rows_v.at[(K - 1) % 3],
                          acc_s.at[dst_v.at[K - 1]], sem_sc).wait()
    pltpu.make_async_copy(y_s.at[src_v.at[K]], rows_v.at[K % 3],
                          gsems[K % 2]).wait()
    pltpu.make_async_copy(y_s.at[src_v.at[K + 1]], rows_v.at[(K + 1) % 3],
                          gsems[(K + 1) % 2]).wait()


def _seg_writeback(p_hbm, acc_s, c, s):
    plsc.subcore_barrier()
    pltpu.sync_copy(acc_s.at[pl.ds(s * ACC_PER_SUB, ACC_PER_SUB)],
                    p_hbm.at[c, pl.ds(s * ACC_PER_SUB, ACC_PER_SUB)])


def _make_seg_body(y_rows):
    rows_per_sub = y_rows // NS

    def _seg_body(y_hbm, src_hbm, dst_hbm, p_hbm, acc_s, y_s, src_v, dst_v,
                  rows_v, zbuf_v, sem_ga, sem_gb, sem_y, sem_sc):
        c = lax.axis_index("c")
        s = lax.axis_index("s")
        wid = c * NS + s
        _seg_common(y_hbm, src_hbm, dst_hbm, acc_s, y_s, src_v, dst_v, rows_v,
                    zbuf_v, sem_ga, sem_gb, sem_y, sem_sc, s, wid,
                    rows_per_sub)
        _seg_writeback(p_hbm, acc_s, c, s)

    return _seg_body


_MESH = dict(core_axis_name="c", subcore_axis_name="s",
             num_cores=NC, num_subcores=NS)


@functools.lru_cache(maxsize=2)
def _seg_kernel(y_rows):
    return pl.kernel(
        _make_seg_body(y_rows),
        out_type=jax.ShapeDtypeStruct((NC, ACC_ROWS, HP), jnp.float32),
        mesh=plsc.VectorSubcoreMesh(**_MESH),
        scratch_types=[
            pltpu.VMEM_SHARED((ACC_ROWS, HP), jnp.float32),
            pltpu.VMEM_SHARED((y_rows, HP), jnp.float32),
            pltpu.VMEM((KP, CH), jnp.int32),
            pltpu.VMEM((KP, CH), jnp.int32),
            pltpu.VMEM((3, CH, HP), jnp.float32),
            pltpu.VMEM((ZROWS, HP), jnp.float32),
            pltpu.SemaphoreType.DMA,
            pltpu.SemaphoreType.DMA,
            pltpu.SemaphoreType.DMA,
            pltpu.SemaphoreType.DMA,
        ],
        compiler_params=pltpu.CompilerParams(use_tc_tiling_on_sc=False),
    )


_XROWS = NN // 4
_ABLK = 1000


def _stage_a_body(x_ref, w_ref, o_ref):
    o = jnp.dot(x_ref[...], w_ref[...], preferred_element_type=jnp.float32)
    o_ref[...] = o.reshape(_ABLK // 8, 8, 128)


def _stage_a(xp, w4):
    grid = (_XROWS // _ABLK,)
    return pl.pallas_call(
        _stage_a_body,
        grid=grid,
        in_specs=[
            pl.BlockSpec((_ABLK, 4 * D_IN), lambda i: (i, 0)),
            pl.BlockSpec((4 * D_IN, 128), lambda i: (0, 0)),
        ],
        out_specs=pl.BlockSpec((_ABLK // 8, 8, 128), lambda i: (i, 0, 0)),
        out_shape=jax.ShapeDtypeStruct((G, 8, 128), jnp.float32),
    )(xp, w4)


def _stage_b1_body(p_ref, r_ref, b_ref, h_ref):
    a = p_ref[0] + p_ref[1]
    h_ref[...] = jnp.maximum(a + r_ref[...] + b_ref[...], 0.0)


def _stage_b1(p, r, b4):
    grid = (G // GB,)
    return pl.pallas_call(
        _stage_b1_body,
        grid=grid,
        in_specs=[
            pl.BlockSpec((NC, GB, 8, 128), lambda i: (0, i, 0, 0)),
            pl.BlockSpec((GB, 8, 128), lambda i: (i, 0, 0)),
            pl.BlockSpec((1, 128), lambda i: (0, 0)),
        ],
        out_specs=pl.BlockSpec((GB, 8, 128), lambda i: (i, 0, 0)),
        out_shape=jax.ShapeDtypeStruct((G, 8, 128), jnp.float32),
    )(p, r, b4)


def _stage_b_body(p_ref, h_ref, wr_ref, wq_ref, b_ref, o_ref):
    a = (p_ref[0] + p_ref[1]).reshape(GB * 8, 128)
    hp = h_ref[...].reshape(GB * 8, 128)
    o = jnp.maximum(
        jnp.dot(a, wr_ref[...], preferred_element_type=jnp.float32)
        + b_ref[...]
        + jnp.dot(hp, wq_ref[...], preferred_element_type=jnp.float32),
        0.0,
    )
    o_ref[...] = o.reshape(GB, 8, 128)


def _stage_b(p, h, wr4, wq4, b4):
    grid = (G // GB,)
    return pl.pallas_call(
        _stage_b_body,
        grid=grid,
        in_specs=[
            pl.BlockSpec((NC, GB, 8, 128), lambda i: (0, i, 0, 0)),
            pl.BlockSpec((GB, 8, 128), lambda i: (i, 0, 0)),
            pl.BlockSpec((128, 128), lambda i: (0, 0)),
            pl.BlockSpec((128, 128), lambda i: (0, 0)),
            pl.BlockSpec((1, 128), lambda i: (0, 0)),
        ],
        out_specs=pl.BlockSpec((GB, 8, 128), lambda i: (i, 0, 0)),
        out_shape=jax.ShapeDtypeStruct((G, 8, 128), jnp.float32),
    )(p, h, wr4, wq4, b4)


_CBLK = 125
_NBLK_C = GG // _CBLK


def _stage_c_body(p_ref, h3_ref, wr_ref, bo_ref, wq_ref, wl1_ref, bl1_ref,
                  wl2_ref, bl2_ref, wh1_ref, bh1_ref, wh2_ref, bh2_ref,
                  wh3_ref, bh3_ref, out_ref, acc):
    i = pl.program_id(0)
    a = (p_ref[0] + p_ref[1]).reshape(_CBLK * 8, 128)
    hp = h3_ref[...].reshape(_CBLK * 8, 128)
    h4 = jnp.maximum(
        jnp.dot(a, wr_ref[...], preferred_element_type=jnp.float32)
        + bo_ref[...]
        + jnp.dot(hp, wq_ref[...], preferred_element_type=jnp.float32),
        0.0,
    )

    rows = _CBLK * 8
    row_iota = lax.broadcasted_iota(jnp.int32, (rows, 1), 0)
    s0 = jnp.zeros((1, D_OUT), jnp.float32)
    s1 = jnp.zeros((1, D_OUT), jnp.float32)
    for q in range(4):
        nid = (i * rows + row_iota) * 4 + q
        hq = h4[:, q * D_OUT:(q + 1) * D_OUT]
        m0 = nid < N
        s0 = s0 + jnp.sum(jnp.where(m0, hq, 0.0), axis=0, keepdims=True)
        s1 = s1 + jnp.sum(jnp.where(m0, 0.0, hq), axis=0, keepdims=True)

    @pl.when(i == 0)
    def _():
        acc[0:1, :] = s0
        acc[1:2, :] = s1

    @pl.when(i > 0)
    def _():
        acc[0:1, :] = acc[0:1, :] + s0
        acc[1:2, :] = acc[1:2, :] + s1

    @pl.when(i == _NBLK_C - 1)
    def _():
        m = jnp.maximum(
            jnp.dot(acc[...], wl1_ref[...], preferred_element_type=jnp.float32)
            + bl1_ref[...], 0.0)
        m = jnp.maximum(
            jnp.dot(m, wl2_ref[...], preferred_element_type=jnp.float32)
            + bl2_ref[...], 0.0)
        z = (jnp.dot(m[0:1, :], wh1_ref[0:D_OUT, :],
                     preferred_element_type=jnp.float32)
             + jnp.dot(m[1:2, :], wh1_ref[D_OUT:2 * D_OUT, :],
                       preferred_element_type=jnp.float32)
             + bh1_ref[...])
        z = jnp.maximum(z, 0.0)
        z = jnp.maximum(
            jnp.dot(z, wh2_ref[...], preferred_element_type=jnp.float32)
            + bh2_ref[...], 0.0)
        z = (jnp.dot(z, wh3_ref[...], preferred_element_type=jnp.float32)
             + bh3_ref[...])
        out_ref[...] = 1.0 / (1.0 + jnp.exp(-z))


def _stage_c(p, h3, wr4, bo4, wq4, wl1, bl1, wl2, bl2, wh1, bh1, wh2, bh2,
             wh3, bh3):
    grid = (_NBLK_C,)

    def _full(shape):
        nd = len(shape)
        return pl.BlockSpec(shape, lambda i, _nd=nd: (0,) * _nd)

    return pl.pallas_call(
        _stage_c_body,
        grid=grid,
        in_specs=[
            pl.BlockSpec((NC, _CBLK, 8, 128), lambda i: (0, i, 0, 0)),
            pl.BlockSpec((_CBLK, 8, 128), lambda i: (i, 0, 0)),
            _full((128, 4 * D_OUT)),
            _full((1, 4 * D_OUT)),
            _full((128, 4 * D_OUT)),
            _full((D_OUT, D_OUT)),
            _full((1, D_OUT)),
            _full((D_OUT, D_OUT)),
            _full((1, D_OUT)),
            _full((2 * D_OUT, 10)),
            _full((1, 10)),
            _full((10, 10)),
            _full((1, 10)),
            _full((10, 1)),
            _full((1, 1)),
        ],
        out_specs=pl.BlockSpec((1, 1), lambda i: (0, 0)),
        out_shape=jax.ShapeDtypeStruct((1, 1), jnp.float32),
        scratch_shapes=[pltpu.VMEM((2, D_OUT), jnp.float32)],
    )(p, h3, wr4, bo4, wq4, wl1, bl1, wl2, bl2, wh1, bh1, wh2, bh2, wh3, bh3)


def _pad_cols(w, width=HP):
    return jnp.pad(w, ((0, 0), (0, width - w.shape[1])))


def _pad_rows(w, height=HP):
    return jnp.pad(w, ((0, height - w.shape[0]), (0, 0)))


def _bd4(w):
    return jnp.kron(jnp.eye(4, dtype=w.dtype), w)


def kernel(mol_1_graph, mol_1_nodes, mol_2_graph, mol_2_nodes, params):
    pr = params
    wr_in4 = _bd4(_pad_cols(pr['conv_in']['W_rel']))
    wq_in4 = _bd4(_pad_cols(pr['conv_in']['W_root']))
    b_in4 = jnp.tile(_pad_cols(pr['conv_in']['b'][None]), (1, 4))
    li1, li2 = pr['conv_internal']
    wr1 = _bd4(_pad_cols(_pad_rows(li1['W_rel'])))
    wq1 = _bd4(_pad_cols(_pad_rows(li1['W_root'])))
    b14 = jnp.tile(_pad_cols(li1['b'][None]), (1, 4))
    wr2 = _bd4(_pad_cols(_pad_rows(li2['W_rel'])))
    wq2 = _bd4(_pad_cols(_pad_rows(li2['W_root'])))
    b24 = jnp.tile(_pad_cols(li2['b'][None]), (1, 4))
    wr_out4 = _bd4(_pad_rows(pr['conv_out']['W_rel']))
    wq_out4 = _bd4(_pad_rows(pr['conv_out']['W_root']))
    b_out4 = jnp.tile(pr['conv_out']['b'][None], (1, 4))
    lo1, lo2 = pr['linear_output']
    wh1 = pr['linear_1']['W']
    bh1 = pr['linear_1']['b'][None]
    wh2 = pr['linear_2']['W']
    bh2 = pr['linear_2']['b'][None]
    wh3 = pr['linear_3']['W']
    bh3 = pr['linear_3']['b'][None]

    xp = jnp.concatenate([mol_1_nodes, mol_2_nodes],
                         axis=0).reshape(_XROWS, 4 * D_IN)
    src = jnp.concatenate([
        mol_1_graph[0], mol_2_graph[0] + N,
        jnp.zeros((EPAD,), jnp.int32),
    ])
    dst = jnp.concatenate([
        mol_1_graph[1], mol_2_graph[1] + N,
        jnp.full((EPAD,), NN, jnp.int32),
    ])
    src3 = jnp.concatenate([
        src.reshape(NW, K, CH),
        jnp.zeros((NW, KP - K, CH), jnp.int32),
    ], axis=1)
    dst3 = jnp.concatenate([
        dst.reshape(NW, K, CH),
        jnp.full((NW, KP - K, CH), NN, jnp.int32),
    ], axis=1)
    t0 = _stage_a(xp, wr_in4)
    p0 = _seg_kernel(ACC_ROWS)(t0.reshape(ACC_ROWS, HP), src3, dst3)
    r0 = _stage_a(xp, wq_in4)
    h1 = _stage_b1(p0.reshape(NC, G, 8, 128), r0, b_in4)
    p1 = _seg_kernel(ACC_ROWS)(h1.reshape(ACC_ROWS, HP), src3, dst3)
    h2 = _stage_b(p1.reshape(NC, G, 8, 128), h1, wr1, wq1, b14)
    p2 = _seg_kernel(ACC_ROWS)(h2.reshape(ACC_ROWS, HP), src3, dst3)
    h3 = _stage_b(p2.reshape(NC, G, 8, 128), h2, wr2, wq2, b24)
    p3 = _seg_kernel(ACC_ROWS)(h3.reshape(ACC_ROWS, HP), src3, dst3)
    out = _stage_c(p3.reshape(NC, G, 8, 128), h3, wr_out4, b_out4, wq_out4,
                   lo1['W'], lo1['b'][None], lo2['W'], lo2['b'][None],
                   wh1, bh1, wh2, bh2, wh3, bh3)
    return out.reshape((1,))


# speedup vs baseline: 7.6376x; 1.0831x over previous
import functools

import jax
import jax.numpy as jnp
from jax import lax
from jax.experimental import pallas as pl
from jax.experimental.pallas import tpu as pltpu
from jax.experimental.pallas import tpu_sc as plsc

N = 10000
E = 160000
D_IN = 256
HID = 20
HP = 32
D_OUT = 128
NN = 2 * N

NC, NS = 2, 16
NW = NC * NS
CH = 128
E2 = 2 * E
K = -(-E2 // (NW * CH))
KP = K + 2
EP = NW * K * CH
EPAD = EP - E2

ACC_ROWS = 20480
ZROWS = 160
ACC_PER_SUB = ACC_ROWS // NS

G = ACC_ROWS * HP // 1024
GB = 64
GG = 625


def _seg_common(y_hbm, src_hbm, dst_hbm, acc_s, y_s, src_v, dst_v, rows_v,
                zbuf_v, sem_ga, sem_gb, sem_y, sem_sc, s, wid,
                y_rows_per_sub):
    pltpu.async_copy(src_hbm.at[wid], src_v, sem_ga)
    pltpu.async_copy(dst_hbm.at[wid], dst_v, sem_gb)
    ysl = pl.ds(s * y_rows_per_sub, y_rows_per_sub)
    pltpu.async_copy(y_hbm.at[ysl], y_s.at[ysl], sem_y)

    zeros16 = jnp.zeros((16,), jnp.float32)

    def _zfill(i, carry):
        zbuf_v[i // 2, pl.ds((i % 2) * 16, 16)] = zeros16
        return carry

    lax.fori_loop(0, ZROWS * 2, _zfill, 0)
    pltpu.make_async_copy(src_hbm.at[wid], src_v, sem_ga).wait()
    pltpu.make_async_copy(dst_hbm.at[wid], dst_v, sem_gb).wait()
    pltpu.make_async_copy(y_hbm.at[ysl], y_s.at[ysl], sem_y).wait()

    zsl = [pl.ds(s * ACC_PER_SUB + t * ZROWS, ZROWS)
           for t in range(ACC_PER_SUB // ZROWS)]
    for t in zsl:
        pltpu.async_copy(zbuf_v, acc_s.at[t], sem_y)
    for t in zsl:
        pltpu.make_async_copy(zbuf_v, acc_s.at[t], sem_y).wait()
    plsc.subcore_barrier()

    gsems = (sem_ga, sem_gb)
    pltpu.async_copy(y_s.at[src_v.at[0]], rows_v.at[0], gsems[0])
    pltpu.async_copy(y_s.at[src_v.at[1]], rows_v.at[1], gsems[1])
    pltpu.make_async_copy(y_s.at[src_v.at[0]], rows_v.at[0], gsems[0]).wait()
    pltpu.async_copy(y_s.at[src_v.at[2]], rows_v.at[2], gsems[0])
    pltpu.async_copy(rows_v.at[0], acc_s.at[dst_v.at[0]], sem_sc, add=True)

    def _six(g, carry):
        for t in range(6):
            j = 1 + 6 * g + t
            b = (1 + t) % 3
            sg = gsems[(1 + t) % 2]
            pb = (b + 2) % 3
            pltpu.make_async_copy(y_s.at[src_v.at[j]], rows_v.at[b],
                                  sg).wait()
            pltpu.make_async_copy(rows_v.at[pb], acc_s.at[dst_v.at[j - 1]],
                                  sem_sc).wait()
            pltpu.async_copy(y_s.at[src_v.at[j + 2]], rows_v.at[pb], sg)
            pltpu.async_copy(rows_v.at[b], acc_s.at[dst_v.at[j]], sem_sc,
                             add=True)
        return carry

    lax.fori_loop(0, (K - 1) // 6, _six, 0)
    pltpu.make_async_copy(rows_v.at[(K - 1) % 3],
                          acc_s.at[dst_v.at[K - 1]], sem_sc).wait()
    pltpu.make_async_copy(y_s.at[src_v.at[K]], rows_v.at[K % 3],
                          gsems[K % 2]).wait()
    pltpu.make_async_copy(y_s.at[src_v.at[K + 1]], rows_v.at[(K + 1) % 3],
                          gsems[(K + 1) % 2]).wait()


def _seg_writeback(p_hbm, acc_s, c, s):
    plsc.subcore_barrier()
    pltpu.sync_copy(acc_s.at[pl.ds(s * ACC_PER_SUB, ACC_PER_SUB)],
                    p_hbm.at[c, pl.ds(s * ACC_PER_SUB, ACC_PER_SUB)])


def _make_seg_body(y_rows):
    rows_per_sub = y_rows // NS

    def _seg_body(y_hbm, src_hbm, dst_hbm, p_hbm, acc_s, y_s, src_v, dst_v,
                  rows_v, zbuf_v, sem_ga, sem_gb, sem_y, sem_sc):
        c = lax.axis_index("c")
        s = lax.axis_index("s")
        wid = c * NS + s
        _seg_common(y_hbm, src_hbm, dst_hbm, acc_s, y_s, src_v, dst_v, rows_v,
                    zbuf_v, sem_ga, sem_gb, sem_y, sem_sc, s, wid,
                    rows_per_sub)
        _seg_writeback(p_hbm, acc_s, c, s)

    return _seg_body


_MESH = dict(core_axis_name="c", subcore_axis_name="s",
             num_cores=NC, num_subcores=NS)


@functools.lru_cache(maxsize=2)
def _seg_kernel(y_rows):
    return pl.kernel(
        _make_seg_body(y_rows),
        out_type=jax.ShapeDtypeStruct((NC, ACC_ROWS, HP), jnp.float32),
        mesh=plsc.VectorSubcoreMesh(**_MESH),
        scratch_types=[
            pltpu.VMEM_SHARED((ACC_ROWS, HP), jnp.float32),
            pltpu.VMEM_SHARED((y_rows, HP), jnp.float32),
            pltpu.VMEM((KP, CH), jnp.int32),
            pltpu.VMEM((KP, CH), jnp.int32),
            pltpu.VMEM((3, CH, HP), jnp.float32),
            pltpu.VMEM((ZROWS, HP), jnp.float32),
            pltpu.SemaphoreType.DMA,
            pltpu.SemaphoreType.DMA,
            pltpu.SemaphoreType.DMA,
            pltpu.SemaphoreType.DMA,
        ],
        compiler_params=pltpu.CompilerParams(use_tc_tiling_on_sc=False),
    )


_XROWS = NN // 4
_ABLK = 1000


def _stage_a_body(x_ref, w_ref, o_ref):
    xp = x_ref[...].reshape(_ABLK, 4 * D_IN)
    o = jnp.dot(xp, w_ref[...], preferred_element_type=jnp.float32)
    o_ref[...] = o.reshape(_ABLK // 8, 8, 128)


def _stage_a(x, w4):
    grid = (_XROWS // _ABLK,)
    return pl.pallas_call(
        _stage_a_body,
        grid=grid,
        in_specs=[
            pl.BlockSpec((4 * _ABLK, D_IN), lambda i: (i, 0)),
            pl.BlockSpec((4 * D_IN, 128), lambda i: (0, 0)),
        ],
        out_specs=pl.BlockSpec((_ABLK // 8, 8, 128), lambda i: (i, 0, 0)),
        out_shape=jax.ShapeDtypeStruct((G, 8, 128), jnp.float32),
    )(x, w4)


def _stage_b1_body(p_ref, r_ref, b_ref, h_ref):
    a = p_ref[0] + p_ref[1]
    h_ref[...] = jnp.maximum(a + r_ref[...] + b_ref[...], 0.0)


def _stage_b1(p, r, b4):
    grid = (G // GB,)
    return pl.pallas_call(
        _stage_b1_body,
        grid=grid,
        in_specs=[
            pl.BlockSpec((NC, GB, 8, 128), lambda i: (0, i, 0, 0)),
            pl.BlockSpec((GB, 8, 128), lambda i: (i, 0, 0)),
            pl.BlockSpec((1, 128), lambda i: (0, 0)),
        ],
        out_specs=pl.BlockSpec((GB, 8, 128), lambda i: (i, 0, 0)),
        out_shape=jax.ShapeDtypeStruct((G, 8, 128), jnp.float32),
    )(p, r, b4)


def _stage_b_body(p_ref, h_ref, wr_ref, wq_ref, b_ref, o_ref):
    a = (p_ref[0] + p_ref[1]).reshape(GB * 8, 128)
    hp = h_ref[...].reshape(GB * 8, 128)
    o = jnp.maximum(
        jnp.dot(a, wr_ref[...], preferred_element_type=jnp.float32)
        + b_ref[...]
        + jnp.dot(hp, wq_ref[...], preferred_element_type=jnp.float32),
        0.0,
    )
    o_ref[...] = o.reshape(GB, 8, 128)


def _stage_b(p, h, wr4, wq4, b4):
    grid = (G // GB,)
    return pl.pallas_call(
        _stage_b_body,
        grid=grid,
        in_specs=[
            pl.BlockSpec((NC, GB, 8, 128), lambda i: (0, i, 0, 0)),
            pl.BlockSpec((GB, 8, 128), lambda i: (i, 0, 0)),
            pl.BlockSpec((128, 128), lambda i: (0, 0)),
            pl.BlockSpec((128, 128), lambda i: (0, 0)),
            pl.BlockSpec((1, 128), lambda i: (0, 0)),
        ],
        out_specs=pl.BlockSpec((GB, 8, 128), lambda i: (i, 0, 0)),
        out_shape=jax.ShapeDtypeStruct((G, 8, 128), jnp.float32),
    )(p, h, wr4, wq4, b4)


_CBLK = 125
_NBLK_C = GG // _CBLK


def _stage_c_body(p_ref, h3_ref, wr_ref, bo_ref, wq_ref, wl1_ref, bl1_ref,
                  wl2_ref, bl2_ref, wh1_ref, bh1_ref, wh2_ref, bh2_ref,
                  wh3_ref, bh3_ref, out_ref, acc):
    i = pl.program_id(0)
    a = (p_ref[0] + p_ref[1]).reshape(_CBLK * 8, 128)
    hp = h3_ref[...].reshape(_CBLK * 8, 128)
    h4 = jnp.maximum(
        jnp.dot(a, wr_ref[...], preferred_element_type=jnp.float32)
        + bo_ref[...]
        + jnp.dot(hp, wq_ref[...], preferred_element_type=jnp.float32),
        0.0,
    )

    rows = _CBLK * 8
    row_iota = lax.broadcasted_iota(jnp.int32, (rows, 1), 0)
    s0 = jnp.zeros((1, D_OUT), jnp.float32)
    s1 = jnp.zeros((1, D_OUT), jnp.float32)
    for q in range(4):
        nid = (i * rows + row_iota) * 4 + q
        hq = h4[:, q * D_OUT:(q + 1) * D_OUT]
        m0 = nid < N
        s0 = s0 + jnp.sum(jnp.where(m0, hq, 0.0), axis=0, keepdims=True)
        s1 = s1 + jnp.sum(jnp.where(m0, 0.0, hq), axis=0, keepdims=True)

    @pl.when(i == 0)
    def _():
        acc[0:1, :] = s0
        acc[1:2, :] = s1

    @pl.when(i > 0)
    def _():
        acc[0:1, :] = acc[0:1, :] + s0
        acc[1:2, :] = acc[1:2, :] + s1

    @pl.when(i == _NBLK_C - 1)
    def _():
        m = jnp.maximum(
            jnp.dot(acc[...], wl1_ref[...], preferred_element_type=jnp.float32)
            + bl1_ref[...], 0.0)
        m = jnp.maximum(
            jnp.dot(m, wl2_ref[...], preferred_element_type=jnp.float32)
            + bl2_ref[...], 0.0)
        z = (jnp.dot(m[0:1, :], wh1_ref[0:D_OUT, :],
                     preferred_element_type=jnp.float32)
             + jnp.dot(m[1:2, :], wh1_ref[D_OUT:2 * D_OUT, :],
                       preferred_element_type=jnp.float32)
             + bh1_ref[...])
        z = jnp.maximum(z, 0.0)
        z = jnp.maximum(
            jnp.dot(z, wh2_ref[...], preferred_element_type=jnp.float32)
            + bh2_ref[...], 0.0)
        z = (jnp.dot(z, wh3_ref[...], preferred_element_type=jnp.float32)
             + bh3_ref[...])
        out_ref[...] = 1.0 / (1.0 + jnp.exp(-z))


def _stage_c(p, h3, wr4, bo4, wq4, wl1, bl1, wl2, bl2, wh1, bh1, wh2, bh2,
             wh3, bh3):
    grid = (_NBLK_C,)

    def _full(shape):
        nd = len(shape)
        return pl.BlockSpec(shape, lambda i, _nd=nd: (0,) * _nd)

    return pl.pallas_call(
        _stage_c_body,
        grid=grid,
        in_specs=[
            pl.BlockSpec((NC, _CBLK, 8, 128), lambda i: (0, i, 0, 0)),
            pl.BlockSpec((_CBLK, 8, 128), lambda i: (i, 0, 0)),
            _full((128, 4 * D_OUT)),
            _full((1, 4 * D_OUT)),
            _full((128, 4 * D_OUT)),
            _full((D_OUT, D_OUT)),
            _full((1, D_OUT)),
            _full((D_OUT, D_OUT)),
            _full((1, D_OUT)),
            _full((2 * D_OUT, 10)),
            _full((1, 10)),
            _full((10, 10)),
            _full((1, 10)),
            _full((10, 1)),
            _full((1, 1)),
        ],
        out_specs=pl.BlockSpec((1, 1), lambda i: (0, 0)),
        out_shape=jax.ShapeDtypeStruct((1, 1), jnp.float32),
        scratch_shapes=[pltpu.VMEM((2, D_OUT), jnp.float32)],
    )(p, h3, wr4, bo4, wq4, wl1, bl1, wl2, bl2, wh1, bh1, wh2, bh2, wh3, bh3)


def _pad_cols(w, width=HP):
    return jnp.pad(w, ((0, 0), (0, width - w.shape[1])))


def _pad_rows(w, height=HP):
    return jnp.pad(w, ((0, height - w.shape[0]), (0, 0)))


def _bd4(w):
    return jnp.kron(jnp.eye(4, dtype=w.dtype), w)


def kernel(mol_1_graph, mol_1_nodes, mol_2_graph, mol_2_nodes, params):
    pr = params
    wr_in4 = _bd4(_pad_cols(pr['conv_in']['W_rel']))
    wq_in4 = _bd4(_pad_cols(pr['conv_in']['W_root']))
    b_in4 = jnp.tile(_pad_cols(pr['conv_in']['b'][None]), (1, 4))
    li1, li2 = pr['conv_internal']
    wr1 = _bd4(_pad_cols(_pad_rows(li1['W_rel'])))
    wq1 = _bd4(_pad_cols(_pad_rows(li1['W_root'])))
    b14 = jnp.tile(_pad_cols(li1['b'][None]), (1, 4))
    wr2 = _bd4(_pad_cols(_pad_rows(li2['W_rel'])))
    wq2 = _bd4(_pad_cols(_pad_rows(li2['W_root'])))
    b24 = jnp.tile(_pad_cols(li2['b'][None]), (1, 4))
    wr_out4 = _bd4(_pad_rows(pr['conv_out']['W_rel']))
    wq_out4 = _bd4(_pad_rows(pr['conv_out']['W_root']))
    b_out4 = jnp.tile(pr['conv_out']['b'][None], (1, 4))
    lo1, lo2 = pr['linear_output']
    wh1 = pr['linear_1']['W']
    bh1 = pr['linear_1']['b'][None]
    wh2 = pr['linear_2']['W']
    bh2 = pr['linear_2']['b'][None]
    wh3 = pr['linear_3']['W']
    bh3 = pr['linear_3']['b'][None]

    xs = jnp.concatenate([mol_1_nodes, mol_2_nodes], axis=0)
    src = jnp.concatenate([
        mol_1_graph[0], mol_2_graph[0] + N,
        jnp.zeros((EPAD,), jnp.int32),
    ])
    dst = jnp.concatenate([
        mol_1_graph[1], mol_2_graph[1] + N,
        jnp.full((EPAD,), NN, jnp.int32),
    ])
    src3 = jnp.concatenate([
        src.reshape(NW, K, CH),
        jnp.zeros((NW, KP - K, CH), jnp.int32),
    ], axis=1)
    dst3 = jnp.concatenate([
        dst.reshape(NW, K, CH),
        jnp.full((NW, KP - K, CH), NN, jnp.int32),
    ], axis=1)
    t0 = _stage_a(xs, wr_in4)
    p0 = _seg_kernel(ACC_ROWS)(t0.reshape(ACC_ROWS, HP), src3, dst3)
    r0 = _stage_a(xs, wq_in4)
    h1 = _stage_b1(p0.reshape(NC, G, 8, 128), r0, b_in4)
    p1 = _seg_kernel(ACC_ROWS)(h1.reshape(ACC_ROWS, HP), src3, dst3)
    h2 = _stage_b(p1.reshape(NC, G, 8, 128), h1, wr1, wq1, b14)
    p2 = _seg_kernel(ACC_ROWS)(h2.reshape(ACC_ROWS, HP), src3, dst3)
    h3 = _stage_b(p2.reshape(NC, G, 8, 128), h2, wr2, wq2, b24)
    p3 = _seg_kernel(ACC_ROWS)(h3.reshape(ACC_ROWS, HP), src3, dst3)
    out = _stage_c(p3.reshape(NC, G, 8, 128), h3, wr_out4, b_out4, wq_out4,
                   lo1['W'], lo1['b'][None], lo2['W'], lo2['b'][None],
                   wh1, bh1, wh2, bh2, wh3, bh3)
    return out.reshape((1,))
